# baseline (speedup 1.0000x reference)
import functools

import jax
import jax.numpy as jnp
from jax import lax
from jax.experimental import pallas as pl
from jax.experimental.pallas import tpu as pltpu
from jax.experimental.pallas import tpu_sc as plsc

N, E, D, H, DE, T = 10000, 320000, 128, 128, 16, 4

NPAD = 10240
CHUNK = 64
GROUP = 16
NCORES, NSUB = 2, 16
NTILES = NCORES * NSUB
CHUNKS_PER_TILE = 160
GROUPS_PER_TILE = CHUNKS_PER_TILE // GROUP
EPT = CHUNK * CHUNKS_PER_TILE
EPAD = NTILES * EPT
NCHUNKS = EPAD // CHUNK
ROWS_PER_TILE = NPAD // NSUB
RB = 1000
EB = 2048


def _dot(a, b):
    return jnp.dot(a, b, preferred_element_type=jnp.float32)


def _tc_pre_body(x_ref, we_ref, be_ref, wmx_ref, wmh_ref, wsx_ref, bu_ref,
                 xin_ref, hpx_ref, hp_ref, sxb_ref):
    xin = jnp.maximum(_dot(x_ref[...], we_ref[...]) + be_ref[...], 0.0)
    xin_ref[...] = xin
    hpx = _dot(xin, wmx_ref[...])
    hpx_ref[...] = hpx
    hp_ref[...] = hpx + _dot(xin, wmh_ref[...])
    sxb_ref[...] = _dot(xin, wsx_ref[...]) + bu_ref[...]


def _tc_pre(x, We, be, Wmx, Wmh, Wsx, bu):
    wspec = pl.BlockSpec((D, H), lambda i: (0, 0))
    bspec = pl.BlockSpec((1, H), lambda i: (0, 0))
    rspec = pl.BlockSpec((RB, D), lambda i: (i, 0))
    return pl.pallas_call(
        _tc_pre_body,
        grid=(N // RB,),
        in_specs=[rspec, wspec, bspec, wspec, wspec, wspec, bspec],
        out_specs=[pl.BlockSpec((RB, H), lambda i: (i, 0))] * 4,
        out_shape=[jax.ShapeDtypeStruct((N, H), jnp.float32)] * 4,
    )(x, We, be, Wmx, Wmh, Wsx, bu)


def _tc_epb_body(ea_ref, we_ref, bm_ref, epb_ref):
    epb_ref[...] = _dot(ea_ref[...], we_ref[...]) + bm_ref[...]


def _tc_epb(ea_p, W_edge, bm):
    return pl.pallas_call(
        _tc_epb_body,
        grid=(EPAD // EB,),
        in_specs=[pl.BlockSpec((EB, DE), lambda i: (i, 0)),
                  pl.BlockSpec((DE, H), lambda i: (0, 0)),
                  pl.BlockSpec((1, H), lambda i: (0, 0))],
        out_specs=pl.BlockSpec((EB, H), lambda i: (i, 0)),
        out_shape=jax.ShapeDtypeStruct((EPAD, H), jnp.float32),
    )(ea_p, W_edge, bm)


def _tc_step_body(aggp_ref, h_ref, hpx_ref, sxb_ref, wu_ref, wsh_ref, wmh_ref,
                  h2_ref, hp2_ref):
    agg = aggp_ref[0] + aggp_ref[1]
    h2 = jnp.maximum(
        _dot(agg, wu_ref[...]) + _dot(h_ref[...], wsh_ref[...]) + sxb_ref[...],
        0.0)
    h2_ref[...] = h2
    hp2_ref[...] = hpx_ref[...] + _dot(h2, wmh_ref[...])


def _tc_step(aggp, h, hpx, sxb, W_upd, Wsh, Wmh):
    wspec = pl.BlockSpec((H, H), lambda i: (0, 0))
    rspec = pl.BlockSpec((RB, H), lambda i: (i, 0))
    return pl.pallas_call(
        _tc_step_body,
        grid=(N // RB,),
        in_specs=[pl.BlockSpec((NCORES, RB, H), lambda i: (0, i, 0)),
                  rspec, rspec, rspec, wspec, wspec, wspec],
        out_specs=[rspec, rspec],
        out_shape=[jax.ShapeDtypeStruct((N, H), jnp.float32)] * 2,
    )(aggp, h, hpx, sxb, W_upd, Wsh, Wmh)


def _tc_last_body(aggp_ref, h_ref, xin_ref, sxb_ref, wu_ref, wsh_ref,
                  wdx_ref, wdh_ref, bd_ref, h2_ref, out_ref):
    agg = aggp_ref[0] + aggp_ref[1]
    h2 = jnp.maximum(
        _dot(agg, wu_ref[...]) + _dot(h_ref[...], wsh_ref[...]) + sxb_ref[...],
        0.0)
    h2_ref[...] = h2
    out_ref[...] = (_dot(xin_ref[...], wdx_ref[...]) +
                    _dot(h2, wdh_ref[...]) + bd_ref[...])


def _tc_last(aggp, h, xin, sxb, W_upd, Wsh, Wdx, Wdh, bd):
    wspec = pl.BlockSpec((H, H), lambda i: (0, 0))
    rspec = pl.BlockSpec((RB, H), lambda i: (i, 0))
    return pl.pallas_call(
        _tc_last_body,
        grid=(N // RB,),
        in_specs=[pl.BlockSpec((NCORES, RB, H), lambda i: (0, i, 0)),
                  rspec, rspec, rspec, wspec, wspec, wspec, wspec,
                  pl.BlockSpec((1, D), lambda i: (0, 0))],
        out_specs=[rspec, pl.BlockSpec((RB, D), lambda i: (i, 0))],
        out_shape=[jax.ShapeDtypeStruct((N, H), jnp.float32),
                   jax.ShapeDtypeStruct((N, D), jnp.float32)],
    )(aggp, h, xin, sxb, W_upd, Wsh, Wdx, Wdh, bd)


def _sc_agg_body(hp_hbm, src_hbm, dst_hbm, epb_hbm, out_hbm,
                 zbuf, srcv0, dstv0, srcv1, dstv1,
                 rows0, epbv0, rows1, epbv1, agg_sh,
                 semg0, seme0, semsc0, semg1, seme1, semsc1, semi0, semi1):
    c = lax.axis_index("c")
    s = lax.axis_index("s")

    rows = (rows0, rows1)
    epbv = (epbv0, epbv1)
    semg = (semg0, semg1)
    seme = (seme0, seme1)
    semsc = (semsc0, semsc1)

    def _z(j, carry):
        for l in range(H // 16):
            zbuf[j, pl.ds(l * 16, 16)] = jnp.zeros((16,), jnp.float32)
        return carry
    lax.fori_loop(0, 8, _z, 0)

    def _zs(k, carry):
        pltpu.sync_copy(zbuf, agg_sh.at[pl.ds(s * ROWS_PER_TILE + k * 8, 8)])
        return carry
    lax.fori_loop(0, ROWS_PER_TILE // 8, _zs, 0)
    plsc.subcore_barrier()

    cbase = c * (NCHUNKS // NCORES) + s * CHUNKS_PER_TILE

    def start_idx(cb, srcv, dstv, semi):
        pltpu.async_copy(src_hbm.at[pl.ds(cb, GROUP)], srcv, semi)
        pltpu.async_copy(dst_hbm.at[pl.ds(cb, GROUP)], dstv, semi)

    def wait_idx(srcv, dstv, semi):
        pltpu.make_async_copy(src_hbm.at[pl.ds(0, GROUP)], srcv, semi).wait()
        pltpu.make_async_copy(dst_hbm.at[pl.ds(0, GROUP)], dstv, semi).wait()

    def compute(r, e):
        @plsc.parallel_loop(0, CHUNK)
        def _(i):
            for l in range(H // 16):
                sl = pl.ds(l * 16, 16)
                r[i, sl] = jnp.maximum(r[i, sl] + e[i, sl], 0.0)

    def run_group(gci, srcv, dstv):
        def start_fetch(j, slot):
            pltpu.async_copy(hp_hbm.at[srcv.at[j]], rows[slot], semg[slot])
            pltpu.async_copy(epb_hbm.at[pl.ds((gci + j) * CHUNK, CHUNK)],
                             epbv[slot], seme[slot])

        def wait_fetch(j, slot):
            pltpu.make_async_copy(hp_hbm.at[srcv.at[j]], rows[slot],
                                  semg[slot]).wait()
            pltpu.make_async_copy(epb_hbm.at[pl.ds(0, CHUNK)], epbv[slot],
                                  seme[slot]).wait()

        def start_scatter(j, slot):
            pltpu.async_copy(rows[slot], agg_sh.at[dstv.at[j]], semsc[slot],
                             add=True)

        def wait_scatter(j, slot):
            pltpu.make_async_copy(rows[slot], agg_sh.at[dstv.at[j]],
                                  semsc[slot]).wait()

        start_fetch(0, 0)
        for p in range(GROUP // 2):
            j0, j1, j2 = 2 * p, 2 * p + 1, 2 * p + 2
            if p > 0:
                wait_scatter(j1 - 2, 1)
            start_fetch(j1, 1)
            wait_fetch(j0, 0)
            compute(rows[0], epbv[0])
            start_scatter(j0, 0)
            wait_fetch(j1, 1)
            compute(rows[1], epbv[1])
            start_scatter(j1, 1)
            wait_scatter(j0, 0)
            if p < GROUP // 2 - 1:
                start_fetch(j2, 0)
        wait_scatter(GROUP - 1, 1)

    start_idx(cbase, srcv0, dstv0, semi0)
    start_idx(cbase + GROUP, srcv1, dstv1, semi1)

    def super_body(sp, carry):
        g0 = cbase + (2 * sp) * GROUP
        g1 = cbase + (2 * sp + 1) * GROUP
        wait_idx(srcv0, dstv0, semi0)
        run_group(g0, srcv0, dstv0)

        @pl.when(sp < GROUPS_PER_TILE // 2 - 1)
        def _():
            start_idx(g0 + 2 * GROUP, srcv0, dstv0, semi0)

        wait_idx(srcv1, dstv1, semi1)
        run_group(g1, srcv1, dstv1)

        @pl.when(sp < GROUPS_PER_TILE // 2 - 1)
        def _():
            start_idx(g1 + 2 * GROUP, srcv1, dstv1, semi1)
        return carry

    lax.fori_loop(0, GROUPS_PER_TILE // 2, super_body, 0)

    plsc.subcore_barrier()
    pltpu.sync_copy(agg_sh.at[pl.ds(s * ROWS_PER_TILE, ROWS_PER_TILE)],
                    out_hbm.at[c, pl.ds(s * ROWS_PER_TILE, ROWS_PER_TILE)])


@functools.cache
def _make_sc_agg():
    return functools.partial(
        pl.kernel,
        out_type=jax.ShapeDtypeStruct((NCORES, NPAD, H), jnp.float32),
        mesh=plsc.VectorSubcoreMesh(core_axis_name="c", subcore_axis_name="s"),
        scratch_types=[
            pltpu.VMEM((8, H), jnp.float32),
            pltpu.VMEM((GROUP, CHUNK), jnp.int32),
            pltpu.VMEM((GROUP, CHUNK), jnp.int32),
            pltpu.VMEM((GROUP, CHUNK), jnp.int32),
            pltpu.VMEM((GROUP, CHUNK), jnp.int32),
            pltpu.VMEM((CHUNK, H), jnp.float32),
            pltpu.VMEM((CHUNK, H), jnp.float32),
            pltpu.VMEM((CHUNK, H), jnp.float32),
            pltpu.VMEM((CHUNK, H), jnp.float32),
            pltpu.VMEM_SHARED((NPAD, H), jnp.float32),
            pltpu.SemaphoreType.DMA,
            pltpu.SemaphoreType.DMA,
            pltpu.SemaphoreType.DMA,
            pltpu.SemaphoreType.DMA,
            pltpu.SemaphoreType.DMA,
            pltpu.SemaphoreType.DMA,
            pltpu.SemaphoreType.DMA,
            pltpu.SemaphoreType.DMA,
        ],
    )(_sc_agg_body)


def kernel(x, edge_index, edge_attr, batch, W_enc, b_enc, W_msg, W_edge, b_msg,
           W_upd, W_self, b_upd, W_dec, b_dec):
    f32 = jnp.float32
    pad = EPAD - E
    src_p = jnp.concatenate([edge_index[0], jnp.zeros((pad,), jnp.int32)])
    dst_p = jnp.concatenate([edge_index[1], jnp.full((pad,), N, jnp.int32)])
    src2 = src_p.reshape(NCHUNKS, CHUNK)
    dst2 = dst_p.reshape(NCHUNKS, CHUNK)
    ea_p = jnp.concatenate([edge_attr, jnp.zeros((pad, DE), f32)])

    Wmx, Wmh = W_msg[:H], W_msg[H:2 * H] + W_msg[2 * H:]
    Wsx, Wsh = W_self[:H], W_self[H:2 * H] + W_self[2 * H:]
    Wdx, Wdh = W_dec[:H], W_dec[H:]
    be, bm = b_enc.reshape(1, H), b_msg.reshape(1, H)
    bu, bd = b_upd.reshape(1, H), b_dec.reshape(1, D)

    xin, hpx, hp, sxb = _tc_pre(x, W_enc, be, Wmx, Wmh, Wsx, bu)
    epb = _tc_epb(ea_p, W_edge, bm)

    sc_agg = _make_sc_agg()
    h = xin
    for _ in range(T - 1):
        aggp = sc_agg(hp, src2, dst2, epb)
        h, hp = _tc_step(aggp, h, hpx, sxb, W_upd, Wsh, Wmh)

    aggp = sc_agg(hp, src2, dst2, epb)
    h, out = _tc_last(aggp, h, xin, sxb, W_upd, Wsh, Wdx, Wdh, bd)
    return (out, h)


# speedup vs baseline: 1.0518x; 1.0518x over previous
import functools

import jax
import jax.numpy as jnp
from jax import lax
from jax.experimental import pallas as pl
from jax.experimental.pallas import tpu as pltpu
from jax.experimental.pallas import tpu_sc as plsc

N, E, D, H, DE, T = 10000, 320000, 128, 128, 16, 4

NPAD = 10240
CHUNK = 64
GROUP = 16
NCORES, NSUB = 2, 16
NTILES = NCORES * NSUB
CHUNKS_PER_TILE = 160
GROUPS_PER_TILE = CHUNKS_PER_TILE // GROUP
EPT = CHUNK * CHUNKS_PER_TILE
EPAD = NTILES * EPT
NCHUNKS = EPAD // CHUNK
ROWS_PER_TILE = NPAD // NSUB
RB = 1000
EB = 2048


def _dot(a, b):
    return jnp.dot(a, b, preferred_element_type=jnp.float32)


def _tc_pre_body(x_ref, we_ref, be_ref, wmx_ref, wmh_ref, wsx_ref, bu_ref,
                 xin_ref, hpx_ref, hp_ref, sxb_ref):
    xin = jnp.maximum(_dot(x_ref[...], we_ref[...]) + be_ref[...], 0.0)
    xin_ref[...] = xin
    hpx = _dot(xin, wmx_ref[...])
    hpx_ref[...] = hpx
    hp_ref[...] = hpx + _dot(xin, wmh_ref[...])
    sxb_ref[...] = _dot(xin, wsx_ref[...]) + bu_ref[...]


def _tc_pre(x, We, be, Wmx, Wmh, Wsx, bu):
    wspec = pl.BlockSpec((D, H), lambda i: (0, 0))
    bspec = pl.BlockSpec((1, H), lambda i: (0, 0))
    rspec = pl.BlockSpec((RB, D), lambda i: (i, 0))
    return pl.pallas_call(
        _tc_pre_body,
        grid=(N // RB,),
        in_specs=[rspec, wspec, bspec, wspec, wspec, wspec, bspec],
        out_specs=[pl.BlockSpec((RB, H), lambda i: (i, 0))] * 4,
        out_shape=[jax.ShapeDtypeStruct((N, H), jnp.float32)] * 4,
    )(x, We, be, Wmx, Wmh, Wsx, bu)


def _tc_epb_body(ea_ref, we_ref, bm_ref, epb_ref):
    epb_ref[...] = _dot(ea_ref[...], we_ref[...]) + bm_ref[...]


def _tc_epb(ea_p, W_edge, bm):
    return pl.pallas_call(
        _tc_epb_body,
        grid=(EPAD // EB,),
        in_specs=[pl.BlockSpec((EB, DE), lambda i: (i, 0)),
                  pl.BlockSpec((DE, H), lambda i: (0, 0)),
                  pl.BlockSpec((1, H), lambda i: (0, 0))],
        out_specs=pl.BlockSpec((EB, H), lambda i: (i, 0)),
        out_shape=jax.ShapeDtypeStruct((EPAD, H), jnp.float32),
    )(ea_p, W_edge, bm)


def _tc_step_body(aggp_ref, h_ref, hpx_ref, sxb_ref, wu_ref, wsh_ref, wmh_ref,
                  h2_ref, hp2_ref):
    agg = aggp_ref[0] + aggp_ref[1]
    h2 = jnp.maximum(
        _dot(agg, wu_ref[...]) + _dot(h_ref[...], wsh_ref[...]) + sxb_ref[...],
        0.0)
    h2_ref[...] = h2
    hp2_ref[...] = hpx_ref[...] + _dot(h2, wmh_ref[...])


def _tc_step(aggp, h, hpx, sxb, W_upd, Wsh, Wmh):
    wspec = pl.BlockSpec((H, H), lambda i: (0, 0))
    rspec = pl.BlockSpec((RB, H), lambda i: (i, 0))
    return pl.pallas_call(
        _tc_step_body,
        grid=(N // RB,),
        in_specs=[pl.BlockSpec((NCORES, RB, H), lambda i: (0, i, 0)),
                  rspec, rspec, rspec, wspec, wspec, wspec],
        out_specs=[rspec, rspec],
        out_shape=[jax.ShapeDtypeStruct((N, H), jnp.float32)] * 2,
    )(aggp, h, hpx, sxb, W_upd, Wsh, Wmh)


def _tc_last_body(aggp_ref, h_ref, xin_ref, sxb_ref, wu_ref, wsh_ref,
                  wdx_ref, wdh_ref, bd_ref, h2_ref, out_ref):
    agg = aggp_ref[0] + aggp_ref[1]
    h2 = jnp.maximum(
        _dot(agg, wu_ref[...]) + _dot(h_ref[...], wsh_ref[...]) + sxb_ref[...],
        0.0)
    h2_ref[...] = h2
    out_ref[...] = (_dot(xin_ref[...], wdx_ref[...]) +
                    _dot(h2, wdh_ref[...]) + bd_ref[...])


def _tc_last(aggp, h, xin, sxb, W_upd, Wsh, Wdx, Wdh, bd):
    wspec = pl.BlockSpec((H, H), lambda i: (0, 0))
    rspec = pl.BlockSpec((RB, H), lambda i: (i, 0))
    return pl.pallas_call(
        _tc_last_body,
        grid=(N // RB,),
        in_specs=[pl.BlockSpec((NCORES, RB, H), lambda i: (0, i, 0)),
                  rspec, rspec, rspec, wspec, wspec, wspec, wspec,
                  pl.BlockSpec((1, D), lambda i: (0, 0))],
        out_specs=[rspec, pl.BlockSpec((RB, D), lambda i: (i, 0))],
        out_shape=[jax.ShapeDtypeStruct((N, H), jnp.float32),
                   jax.ShapeDtypeStruct((N, D), jnp.float32)],
    )(aggp, h, xin, sxb, W_upd, Wsh, Wdx, Wdh, bd)


def _sc_agg_body(hp_hbm, src_hbm, dst_hbm, epb_hbm, out_hbm,
                 zbuf, srcv, dstv, gidx0, sidx0, gidx1, sidx1,
                 rows0, epbv0, rows1, epbv1, agg_sh,
                 semg0, seme0, semsc0, semg1, seme1, semsc1, semi, semz):
    c = lax.axis_index("c")
    s = lax.axis_index("s")

    def _z(j, carry):
        for l in range(H // 16):
            zbuf[j, pl.ds(l * 16, 16)] = jnp.zeros((16,), jnp.float32)
        return carry
    lax.fori_loop(0, 32, _z, 0)

    def _zs(k, carry):
        pltpu.async_copy(zbuf, agg_sh.at[pl.ds(s * ROWS_PER_TILE + k * 32, 32)],
                         semz)
        return carry
    lax.fori_loop(0, ROWS_PER_TILE // 32, _zs, 0)

    def _zw(k, carry):
        pltpu.make_async_copy(zbuf, agg_sh.at[pl.ds(s * ROWS_PER_TILE, 32)],
                              semz).wait()
        return carry
    lax.fori_loop(0, ROWS_PER_TILE // 32, _zw, 0)
    plsc.subcore_barrier()

    cbase = c * (NCHUNKS // NCORES) + s * CHUNKS_PER_TILE

    def start_idx(g):
        half = pl.ds(pl.multiple_of(lax.rem(g, 32), GROUP), GROUP)
        off = pl.multiple_of(cbase + g, GROUP)
        pltpu.async_copy(src_hbm.at[pl.ds(off, GROUP)], srcv.at[half], semi)
        pltpu.async_copy(dst_hbm.at[pl.ds(off, GROUP)], dstv.at[half], semi)

    def wait_idx():
        half = pl.ds(0, GROUP)
        pltpu.make_async_copy(src_hbm.at[pl.ds(cbase, GROUP)], srcv.at[half],
                              semi).wait()
        pltpu.make_async_copy(dst_hbm.at[pl.ds(cbase, GROUP)], dstv.at[half],
                              semi).wait()

    def start_fetch(g, gidx, sidx, rows, epbv, semg, seme):
        j = lax.rem(g, 32)
        for l in range(CHUNK // 16):
            sl = pl.ds(l * 16, 16)
            gidx[sl] = srcv[j, sl]
            sidx[sl] = dstv[j, sl]
        pltpu.async_copy(hp_hbm.at[gidx], rows, semg)
        eoff = pl.multiple_of((cbase + g) * CHUNK, CHUNK)
        pltpu.async_copy(epb_hbm.at[pl.ds(eoff, CHUNK)], epbv, seme)

    def wait_fetch(gidx, rows, epbv, semg, seme):
        pltpu.make_async_copy(hp_hbm.at[gidx], rows, semg).wait()
        pltpu.make_async_copy(epb_hbm.at[pl.ds(0, CHUNK)], epbv, seme).wait()

    def compute(r, e):
        @plsc.parallel_loop(0, CHUNK)
        def _(i):
            for l in range(H // 16):
                sl = pl.ds(l * 16, 16)
                r[i, sl] = jnp.maximum(r[i, sl] + e[i, sl], 0.0)

    def start_scatter(sidx, rows, semsc):
        pltpu.async_copy(rows, agg_sh.at[sidx], semsc, add=True)

    def wait_scatter(sidx, rows, semsc):
        pltpu.make_async_copy(rows, agg_sh.at[sidx], semsc).wait()

    start_idx(0)
    wait_idx()
    start_idx(GROUP)
    start_fetch(0, gidx0, sidx0, rows0, epbv0, semg0, seme0)

    def pair(p, carry):
        g = 2 * p

        @pl.when(p > 0)
        def _():
            wait_scatter(sidx1, rows1, semsc1)

            @pl.when(jnp.logical_and(lax.rem(g, GROUP) == 0,
                                     g + GROUP < CHUNKS_PER_TILE))
            def _():
                start_idx(g + GROUP)

        start_fetch(g + 1, gidx1, sidx1, rows1, epbv1, semg1, seme1)
        wait_fetch(gidx0, rows0, epbv0, semg0, seme0)
        compute(rows0, epbv0)
        start_scatter(sidx0, rows0, semsc0)
        wait_fetch(gidx1, rows1, epbv1, semg1, seme1)
        compute(rows1, epbv1)
        start_scatter(sidx1, rows1, semsc1)
        wait_scatter(sidx0, rows0, semsc0)

        @pl.when(p < CHUNKS_PER_TILE // 2 - 1)
        def _():
            @pl.when(lax.rem(g + 2, GROUP) == 0)
            def _():
                wait_idx()
            start_fetch(g + 2, gidx0, sidx0, rows0, epbv0, semg0, seme0)
        return carry

    lax.fori_loop(0, CHUNKS_PER_TILE // 2, pair, 0)
    wait_scatter(sidx1, rows1, semsc1)

    plsc.subcore_barrier()
    pltpu.sync_copy(agg_sh.at[pl.ds(s * ROWS_PER_TILE, ROWS_PER_TILE)],
                    out_hbm.at[c, pl.ds(s * ROWS_PER_TILE, ROWS_PER_TILE)])


@functools.cache
def _make_sc_agg():
    return functools.partial(
        pl.kernel,
        out_type=jax.ShapeDtypeStruct((NCORES, NPAD, H), jnp.float32),
        mesh=plsc.VectorSubcoreMesh(core_axis_name="c", subcore_axis_name="s"),
        scratch_types=[
            pltpu.VMEM((32, H), jnp.float32),
            pltpu.VMEM((2 * GROUP, CHUNK), jnp.int32),
            pltpu.VMEM((2 * GROUP, CHUNK), jnp.int32),
            pltpu.VMEM((CHUNK,), jnp.int32),
            pltpu.VMEM((CHUNK,), jnp.int32),
            pltpu.VMEM((CHUNK,), jnp.int32),
            pltpu.VMEM((CHUNK,), jnp.int32),
            pltpu.VMEM((CHUNK, H), jnp.float32),
            pltpu.VMEM((CHUNK, H), jnp.float32),
            pltpu.VMEM((CHUNK, H), jnp.float32),
            pltpu.VMEM((CHUNK, H), jnp.float32),
            pltpu.VMEM_SHARED((NPAD, H), jnp.float32),
            pltpu.SemaphoreType.DMA,
            pltpu.SemaphoreType.DMA,
            pltpu.SemaphoreType.DMA,
            pltpu.SemaphoreType.DMA,
            pltpu.SemaphoreType.DMA,
            pltpu.SemaphoreType.DMA,
            pltpu.SemaphoreType.DMA,
            pltpu.SemaphoreType.DMA,
        ],
    )(_sc_agg_body)


def kernel(x, edge_index, edge_attr, batch, W_enc, b_enc, W_msg, W_edge, b_msg,
           W_upd, W_self, b_upd, W_dec, b_dec):
    f32 = jnp.float32
    pad = EPAD - E
    src_p = jnp.concatenate([edge_index[0], jnp.zeros((pad,), jnp.int32)])
    dst_p = jnp.concatenate([edge_index[1], jnp.full((pad,), N, jnp.int32)])
    src2 = src_p.reshape(NCHUNKS, CHUNK)
    dst2 = dst_p.reshape(NCHUNKS, CHUNK)
    ea_p = jnp.concatenate([edge_attr, jnp.zeros((pad, DE), f32)])

    Wmx, Wmh = W_msg[:H], W_msg[H:2 * H] + W_msg[2 * H:]
    Wsx, Wsh = W_self[:H], W_self[H:2 * H] + W_self[2 * H:]
    Wdx, Wdh = W_dec[:H], W_dec[H:]
    be, bm = b_enc.reshape(1, H), b_msg.reshape(1, H)
    bu, bd = b_upd.reshape(1, H), b_dec.reshape(1, D)

    xin, hpx, hp, sxb = _tc_pre(x, W_enc, be, Wmx, Wmh, Wsx, bu)
    epb = _tc_epb(ea_p, W_edge, bm)

    sc_agg = _make_sc_agg()
    h = xin
    for _ in range(T - 1):
        aggp = sc_agg(hp, src2, dst2, epb)
        h, hp = _tc_step(aggp, h, hpx, sxb, W_upd, Wsh, Wmh)

    aggp = sc_agg(hp, src2, dst2, epb)
    h, out = _tc_last(aggp, h, xin, sxb, W_upd, Wsh, Wdx, Wdh, bd)
    return (out, h)


# speedup vs baseline: 1.1192x; 1.0641x over previous
import functools

import jax
import jax.numpy as jnp
from jax import lax
from jax.experimental import pallas as pl
from jax.experimental.pallas import tpu as pltpu
from jax.experimental.pallas import tpu_sc as plsc

N, E, D, H, DE, T = 10000, 320000, 128, 128, 16, 4

NPAD = 10240
CHUNK = 64
GROUP = 16
NCORES, NSUB = 2, 16
NTILES = NCORES * NSUB
CHUNKS_PER_TILE = 160
EPT = CHUNK * CHUNKS_PER_TILE
EPAD = NTILES * EPT
NCHUNKS = EPAD // CHUNK
ROWS_PER_TILE = NPAD // NSUB
RB = 1000
EB = 2048
EA_PAD = -(-E // EB) * EB

def _dot(a, b):
    return jnp.dot(a, b, preferred_element_type=jnp.float32)


def _tc_pre_body(x_ref, we_ref, be_ref, wmx_ref, wmh_ref, wsx_ref, bu_ref,
                 xin_ref, hpx_ref, hp_ref, sxb_ref):
    xin = jnp.maximum(_dot(x_ref[...], we_ref[...]) + be_ref[...], 0.0)
    xin_ref[...] = xin
    hpx = _dot(xin, wmx_ref[...])
    hpx_ref[...] = hpx
    hp_ref[...] = hpx + _dot(xin, wmh_ref[...])
    sxb_ref[...] = _dot(xin, wsx_ref[...]) + bu_ref[...]


def _tc_pre(x, We, be, Wmx, Wmh, Wsx, bu):
    wspec = pl.BlockSpec((D, H), lambda i: (0, 0))
    bspec = pl.BlockSpec((1, H), lambda i: (0, 0))
    rspec = pl.BlockSpec((RB, D), lambda i: (i, 0))
    ospec = pl.BlockSpec((RB, H), lambda i: (i, 0))
    return pl.pallas_call(
        _tc_pre_body,
        grid=(N // RB,),
        in_specs=[rspec, wspec, bspec, wspec, wspec, wspec, bspec],
        out_specs=[ospec] * 4,
        out_shape=[jax.ShapeDtypeStruct((N, H), jnp.float32)] * 4,
    )(x, We, be, Wmx, Wmh, Wsx, bu)


def _tc_epb_body(ea_ref, we_ref, bm_ref, epb_ref):
    epb_ref[...] = _dot(ea_ref[...], we_ref[...]) + bm_ref[...]


def _tc_epb(ea_p, W_edge, bm):
    return pl.pallas_call(
        _tc_epb_body,
        grid=(EA_PAD // EB,),
        in_specs=[pl.BlockSpec((EB, DE), lambda i: (i, 0)),
                  pl.BlockSpec((DE, H), lambda i: (0, 0)),
                  pl.BlockSpec((1, H), lambda i: (0, 0))],
        out_specs=pl.BlockSpec((EB, H), lambda i: (i, 0)),
        out_shape=jax.ShapeDtypeStruct((EPAD, H), jnp.float32),
    )(ea_p, W_edge, bm)


def _tc_step_body(aggp_ref, h_ref, hpx_ref, sxb_ref, wu_ref, wsh_ref, wmh_ref,
                  h2_ref, hp2_ref):
    agg = aggp_ref[0] + aggp_ref[1]
    h2 = jnp.maximum(
        _dot(agg, wu_ref[...]) + _dot(h_ref[...], wsh_ref[...]) + sxb_ref[...],
        0.0)
    h2_ref[...] = h2
    hp2_ref[...] = hpx_ref[...] + _dot(h2, wmh_ref[...])


def _tc_step(aggp, h, hpx, sxb, W_upd, Wsh, Wmh):
    wspec = pl.BlockSpec((H, H), lambda i: (0, 0))
    rspec = pl.BlockSpec((RB, H), lambda i: (i, 0))
    return pl.pallas_call(
        _tc_step_body,
        grid=(N // RB,),
        in_specs=[pl.BlockSpec((NCORES, RB, H), lambda i: (0, i, 0)),
                  rspec, rspec, rspec, wspec, wspec, wspec],
        out_specs=[rspec, rspec],
        out_shape=[jax.ShapeDtypeStruct((N, H), jnp.float32)] * 2,
    )(aggp, h, hpx, sxb, W_upd, Wsh, Wmh)


def _tc_last_body(aggp_ref, h_ref, xin_ref, sxb_ref, wu_ref, wsh_ref,
                  wdx_ref, wdh_ref, bd_ref, h2_ref, out_ref):
    agg = aggp_ref[0] + aggp_ref[1]
    h2 = jnp.maximum(
        _dot(agg, wu_ref[...]) + _dot(h_ref[...], wsh_ref[...]) + sxb_ref[...],
        0.0)
    h2_ref[...] = h2
    out_ref[...] = (_dot(xin_ref[...], wdx_ref[...]) +
                    _dot(h2, wdh_ref[...]) + bd_ref[...])


def _tc_last(aggp, h, xin, sxb, W_upd, Wsh, Wdx, Wdh, bd):
    wspec = pl.BlockSpec((H, H), lambda i: (0, 0))
    rspec = pl.BlockSpec((RB, H), lambda i: (i, 0))
    return pl.pallas_call(
        _tc_last_body,
        grid=(N // RB,),
        in_specs=[pl.BlockSpec((NCORES, RB, H), lambda i: (0, i, 0)),
                  rspec, rspec, rspec, wspec, wspec, wspec, wspec,
                  pl.BlockSpec((1, D), lambda i: (0, 0))],
        out_specs=[rspec, pl.BlockSpec((RB, D), lambda i: (i, 0))],
        out_shape=[jax.ShapeDtypeStruct((N, H), jnp.float32),
                   jax.ShapeDtypeStruct((N, D), jnp.float32)],
    )(aggp, h, xin, sxb, W_upd, Wsh, Wdx, Wdh, bd)


def _sc_agg_body(hp_hbm, src_hbm, dst_hbm, epb_hbm, out_hbm,
                 zbuf, srcv, dstv, gidx0, sidx0, gidx1, sidx1,
                 rows0, epbv0, rows1, epbv1, agg_sh,
                 semg0, seme0, semsc0, semg1, seme1, semsc1, semi, semz):
    c = lax.axis_index("c")
    s = lax.axis_index("s")

    def _z(j, carry):
        for l in range(H // 16):
            zbuf[j, pl.ds(l * 16, 16)] = jnp.zeros((16,), jnp.float32)
        return carry
    lax.fori_loop(0, 32, _z, 0)

    def _zs(k, carry):
        pltpu.async_copy(zbuf, agg_sh.at[pl.ds(s * ROWS_PER_TILE + k * 32, 32)],
                         semz)
        return carry
    lax.fori_loop(0, ROWS_PER_TILE // 32, _zs, 0)

    def _zw(k, carry):
        pltpu.make_async_copy(zbuf, agg_sh.at[pl.ds(s * ROWS_PER_TILE, 32)],
                              semz).wait()
        return carry
    lax.fori_loop(0, ROWS_PER_TILE // 32, _zw, 0)
    plsc.subcore_barrier()

    cbase = c * (NCHUNKS // NCORES) + s * CHUNKS_PER_TILE

    def start_idx(g):
        half = pl.ds(pl.multiple_of(lax.rem(g, 32), GROUP), GROUP)
        off = pl.multiple_of(cbase + g, GROUP)
        pltpu.async_copy(src_hbm.at[pl.ds(off, GROUP)], srcv.at[half], semi)
        pltpu.async_copy(dst_hbm.at[pl.ds(off, GROUP)], dstv.at[half], semi)

    def wait_idx():
        half = pl.ds(0, GROUP)
        pltpu.make_async_copy(src_hbm.at[pl.ds(cbase, GROUP)], srcv.at[half],
                              semi).wait()
        pltpu.make_async_copy(dst_hbm.at[pl.ds(cbase, GROUP)], dstv.at[half],
                              semi).wait()

    def start_gather(g, gidx, rows, semg):
        j = lax.rem(g, 32)
        for l in range(CHUNK // 16):
            sl = pl.ds(l * 16, 16)
            gidx[sl] = srcv[j, sl]
        pltpu.async_copy(hp_hbm.at[gidx], rows, semg)

    def start_epb(g, epbv, seme):
        eoff = pl.multiple_of((cbase + g) * CHUNK, CHUNK)
        pltpu.async_copy(epb_hbm.at[pl.ds(eoff, CHUNK)], epbv, seme)

    def wait_gather(gidx, rows, semg):
        pltpu.make_async_copy(hp_hbm.at[gidx], rows, semg).wait()

    def wait_epb(epbv, seme):
        pltpu.make_async_copy(epb_hbm.at[pl.ds(0, CHUNK)], epbv, seme).wait()

    def compute(r, e):
        @plsc.parallel_loop(0, CHUNK)
        def _(i):
            for l in range(H // 16):
                sl = pl.ds(l * 16, 16)
                e[i, sl] = jnp.maximum(r[i, sl] + e[i, sl], 0.0)

    def start_scatter(g, sidx, m, semsc):
        j = lax.rem(g, 32)
        for l in range(CHUNK // 16):
            sl = pl.ds(l * 16, 16)
            sidx[sl] = dstv[j, sl]
        pltpu.async_copy(m, agg_sh.at[sidx], semsc, add=True)

    def wait_scatter(sidx, m, semsc):
        pltpu.make_async_copy(m, agg_sh.at[sidx], semsc).wait()

    start_idx(0)
    wait_idx()
    start_idx(GROUP)
    start_gather(0, gidx0, rows0, semg0)
    start_epb(0, epbv0, seme0)

    def pair(p, carry):
        g = 2 * p

        @pl.when(jnp.logical_and(p > 0,
                                 jnp.logical_and(lax.rem(g, GROUP) == 0,
                                                 g + GROUP < CHUNKS_PER_TILE)))
        def _():
            start_idx(g + GROUP)

        start_gather(g + 1, gidx1, rows1, semg1)

        @pl.when(p > 0)
        def _():
            wait_scatter(sidx1, epbv1, semsc1)
        start_epb(g + 1, epbv1, seme1)

        wait_gather(gidx0, rows0, semg0)
        wait_epb(epbv0, seme0)
        compute(rows0, epbv0)
        start_scatter(g, sidx0, epbv0, semsc0)

        @pl.when(p < CHUNKS_PER_TILE // 2 - 1)
        def _():
            @pl.when(lax.rem(g + 2, GROUP) == 0)
            def _():
                wait_idx()
            start_gather(g + 2, gidx0, rows0, semg0)

        wait_gather(gidx1, rows1, semg1)
        wait_epb(epbv1, seme1)
        compute(rows1, epbv1)
        start_scatter(g + 1, sidx1, epbv1, semsc1)

        @pl.when(p < CHUNKS_PER_TILE // 2 - 1)
        def _():
            wait_scatter(sidx0, epbv0, semsc0)
            start_epb(g + 2, epbv0, seme0)
        return carry

    lax.fori_loop(0, CHUNKS_PER_TILE // 2, pair, 0)
    wait_scatter(sidx0, epbv0, semsc0)
    wait_scatter(sidx1, epbv1, semsc1)

    plsc.subcore_barrier()
    pltpu.sync_copy(agg_sh.at[pl.ds(s * ROWS_PER_TILE, ROWS_PER_TILE)],
                    out_hbm.at[c, pl.ds(s * ROWS_PER_TILE, ROWS_PER_TILE)])


@functools.cache
def _make_sc_agg():
    return functools.partial(
        pl.kernel,
        out_type=jax.ShapeDtypeStruct((NCORES, NPAD, H), jnp.float32),
        mesh=plsc.VectorSubcoreMesh(core_axis_name="c", subcore_axis_name="s"),
        scratch_types=[
            pltpu.VMEM((32, H), jnp.float32),
            pltpu.VMEM((2 * GROUP, CHUNK), jnp.int32),
            pltpu.VMEM((2 * GROUP, CHUNK), jnp.int32),
            pltpu.VMEM((CHUNK,), jnp.int32),
            pltpu.VMEM((CHUNK,), jnp.int32),
            pltpu.VMEM((CHUNK,), jnp.int32),
            pltpu.VMEM((CHUNK,), jnp.int32),
            pltpu.VMEM((CHUNK, H), jnp.float32),
            pltpu.VMEM((CHUNK, H), jnp.float32),
            pltpu.VMEM((CHUNK, H), jnp.float32),
            pltpu.VMEM((CHUNK, H), jnp.float32),
            pltpu.VMEM_SHARED((NPAD, H), jnp.float32),
            pltpu.SemaphoreType.DMA,
            pltpu.SemaphoreType.DMA,
            pltpu.SemaphoreType.DMA,
            pltpu.SemaphoreType.DMA,
            pltpu.SemaphoreType.DMA,
            pltpu.SemaphoreType.DMA,
            pltpu.SemaphoreType.DMA,
            pltpu.SemaphoreType.DMA,
        ],
    )(_sc_agg_body)


def kernel(x, edge_index, edge_attr, batch, W_enc, b_enc, W_msg, W_edge, b_msg,
           W_upd, W_self, b_upd, W_dec, b_dec):
    f32 = jnp.float32
    pad = EPAD - E
    src_p = jnp.concatenate([edge_index[0], jnp.zeros((pad,), jnp.int32)])
    dst_p = jnp.concatenate([edge_index[1], jnp.full((pad,), N, jnp.int32)])
    src2 = src_p.reshape(NCHUNKS, CHUNK)
    dst2 = dst_p.reshape(NCHUNKS, CHUNK)
    ea_p = jnp.concatenate([edge_attr, jnp.zeros((EA_PAD - E, DE), f32)])

    Wmx, Wmh = W_msg[:H], W_msg[H:2 * H] + W_msg[2 * H:]
    Wsx, Wsh = W_self[:H], W_self[H:2 * H] + W_self[2 * H:]
    Wdx, Wdh = W_dec[:H], W_dec[H:]
    be, bm = b_enc.reshape(1, H), b_msg.reshape(1, H)
    bu, bd = b_upd.reshape(1, H), b_dec.reshape(1, D)

    xin, hpx, hp, sxb = _tc_pre(x, W_enc, be, Wmx, Wmh, Wsx, bu)
    epb = _tc_epb(ea_p, W_edge, bm)

    sc_agg = _make_sc_agg()
    h = xin
    for _ in range(T - 1):
        aggp = sc_agg(hp, src2, dst2, epb)
        h, hp = _tc_step(aggp, h, hpx, sxb, W_upd, Wsh, Wmh)

    aggp = sc_agg(hp, src2, dst2, epb)
    h, out = _tc_last(aggp, h, xin, sxb, W_upd, Wsh, Wdx, Wdh, bd)
    return (out, h)


# speedup vs baseline: 1.3898x; 1.2418x over previous
import functools

import jax
import jax.numpy as jnp
from jax import lax
from jax.experimental import pallas as pl
from jax.experimental.pallas import tpu as pltpu
from jax.experimental.pallas import tpu_sc as plsc

N, E, D, H, DE, T = 10000, 320000, 128, 128, 16, 4

NPAD = 10240
CHUNK = 64
GROUP = 16
NCORES, NSUB = 2, 16
NTILES = NCORES * NSUB
NC0, NC1 = 184, 132
EPAD = NSUB * (NC0 + NC1) * CHUNK
ROWS_PER_TILE = NPAD // NSUB
RB = 1000
EB = 2048
EA_PAD = -(-E // EB) * EB

def _dot(a, b):
    return jnp.dot(a, b, preferred_element_type=jnp.float32)


def _tc_pre_body(x_ref, we_ref, be_ref, wmx_ref, wmh_ref, wsx_ref, bu_ref,
                 xin_ref, hpx_ref, hp_ref, sxb_ref):
    xin = jnp.maximum(_dot(x_ref[...], we_ref[...]) + be_ref[...], 0.0)
    xin_ref[...] = xin
    hpx = _dot(xin, wmx_ref[...])
    hpx_ref[...] = hpx
    hp_ref[...] = hpx + _dot(xin, wmh_ref[...])
    sxb_ref[...] = _dot(xin, wsx_ref[...]) + bu_ref[...]


def _tc_pre(x, We, be, Wmx, Wmh, Wsx, bu):
    wspec = pl.BlockSpec((D, H), lambda i: (0, 0))
    bspec = pl.BlockSpec((1, H), lambda i: (0, 0))
    rspec = pl.BlockSpec((RB, D), lambda i: (i, 0))
    ospec = pl.BlockSpec((RB, H), lambda i: (i, 0))
    return pl.pallas_call(
        _tc_pre_body,
        grid=(N // RB,),
        in_specs=[rspec, wspec, bspec, wspec, wspec, wspec, bspec],
        out_specs=[ospec] * 4,
        out_shape=[jax.ShapeDtypeStruct((N, H), jnp.float32)] * 4,
    )(x, We, be, Wmx, Wmh, Wsx, bu)


def _tc_epb_body(ea_ref, we_ref, bm_ref, epb_ref):
    epb_ref[...] = _dot(ea_ref[...], we_ref[...]) + bm_ref[...]


def _tc_epb(ea_p, W_edge, bm):
    return pl.pallas_call(
        _tc_epb_body,
        grid=(EA_PAD // EB,),
        in_specs=[pl.BlockSpec((EB, DE), lambda i: (i, 0)),
                  pl.BlockSpec((DE, H), lambda i: (0, 0)),
                  pl.BlockSpec((1, H), lambda i: (0, 0))],
        out_specs=pl.BlockSpec((EB, H), lambda i: (i, 0)),
        out_shape=jax.ShapeDtypeStruct((EPAD, H), jnp.float32),
    )(ea_p, W_edge, bm)


def _tc_step_body(aggp_ref, h_ref, hpx_ref, sxb_ref, wu_ref, wsh_ref, wmh_ref,
                  h2_ref, hp2_ref):
    agg = aggp_ref[0] + aggp_ref[1]
    h2 = jnp.maximum(
        _dot(agg, wu_ref[...]) + _dot(h_ref[...], wsh_ref[...]) + sxb_ref[...],
        0.0)
    h2_ref[...] = h2
    hp2_ref[...] = hpx_ref[...] + _dot(h2, wmh_ref[...])


def _tc_step(aggp, h, hpx, sxb, W_upd, Wsh, Wmh):
    wspec = pl.BlockSpec((H, H), lambda i: (0, 0))
    rspec = pl.BlockSpec((RB, H), lambda i: (i, 0))
    return pl.pallas_call(
        _tc_step_body,
        grid=(N // RB,),
        in_specs=[pl.BlockSpec((NCORES, RB, H), lambda i: (0, i, 0)),
                  rspec, rspec, rspec, wspec, wspec, wspec],
        out_specs=[rspec, rspec],
        out_shape=[jax.ShapeDtypeStruct((N, H), jnp.float32)] * 2,
    )(aggp, h, hpx, sxb, W_upd, Wsh, Wmh)


def _tc_last_body(aggp_ref, h_ref, xin_ref, sxb_ref, wu_ref, wsh_ref,
                  wdx_ref, wdh_ref, bd_ref, h2_ref, out_ref):
    agg = aggp_ref[0] + aggp_ref[1]
    h2 = jnp.maximum(
        _dot(agg, wu_ref[...]) + _dot(h_ref[...], wsh_ref[...]) + sxb_ref[...],
        0.0)
    h2_ref[...] = h2
    out_ref[...] = (_dot(xin_ref[...], wdx_ref[...]) +
                    _dot(h2, wdh_ref[...]) + bd_ref[...])


def _tc_last(aggp, h, xin, sxb, W_upd, Wsh, Wdx, Wdh, bd):
    wspec = pl.BlockSpec((H, H), lambda i: (0, 0))
    rspec = pl.BlockSpec((RB, H), lambda i: (i, 0))
    return pl.pallas_call(
        _tc_last_body,
        grid=(N // RB,),
        in_specs=[pl.BlockSpec((NCORES, RB, H), lambda i: (0, i, 0)),
                  rspec, rspec, rspec, wspec, wspec, wspec, wspec,
                  pl.BlockSpec((1, D), lambda i: (0, 0))],
        out_specs=[rspec, pl.BlockSpec((RB, D), lambda i: (i, 0))],
        out_shape=[jax.ShapeDtypeStruct((N, H), jnp.float32),
                   jax.ShapeDtypeStruct((N, D), jnp.float32)],
    )(aggp, h, xin, sxb, W_upd, Wsh, Wdx, Wdh, bd)


def _sc_agg_body(hp_hbm, src_hbm, dst_hbm, epb_hbm, out_hbm,
                 zbuf, idxs0, idxd0, idxs1, idxd1,
                 rows0, epbv0, rows1, epbv1, agg_sh,
                 semg0, seme0, semsc0, semg1, seme1, semsc1, semz):
    c = lax.axis_index("c")
    s = lax.axis_index("s")

    def _z(j, carry):
        for l in range(H // 16):
            zbuf[j, pl.ds(l * 16, 16)] = jnp.zeros((16,), jnp.float32)
        return carry
    lax.fori_loop(0, 32, _z, 0)

    def _zs(k, carry):
        pltpu.async_copy(zbuf, agg_sh.at[pl.ds(s * ROWS_PER_TILE + k * 32, 32)],
                         semz)
        return carry
    lax.fori_loop(0, ROWS_PER_TILE // 32, _zs, 0)

    def _zw(k, carry):
        pltpu.make_async_copy(zbuf, agg_sh.at[pl.ds(s * ROWS_PER_TILE, 32)],
                              semz).wait()
        return carry
    lax.fori_loop(0, ROWS_PER_TILE // 32, _zw, 0)
    plsc.subcore_barrier()

    cpt = jnp.where(c == 0, NC0, NC1)
    base = c * (NSUB * NC0 * CHUNK) + s * cpt * CHUNK

    def load_idx(eb, is_, id_):
        pltpu.sync_copy(src_hbm.at[pl.ds(eb, CHUNK)], is_)
        pltpu.sync_copy(dst_hbm.at[pl.ds(eb, CHUNK)], id_)

    def start_fetch(eb, is_, rows, epbv, semg, seme):
        pltpu.async_copy(hp_hbm.at[is_], rows, semg)
        pltpu.async_copy(epb_hbm.at[pl.ds(eb, CHUNK)], epbv, seme)

    def wait_fetch(is_, rows, epbv, semg, seme):
        pltpu.make_async_copy(hp_hbm.at[is_], rows, semg).wait()
        pltpu.make_async_copy(epb_hbm.at[pl.ds(0, CHUNK)], epbv, seme).wait()

    def compute(r, e):
        @plsc.parallel_loop(0, CHUNK)
        def _(i):
            for l in range(H // 16):
                sl = pl.ds(l * 16, 16)
                r[i, sl] = jnp.maximum(r[i, sl] + e[i, sl], 0.0)

    def start_scatter(rows, id_, semsc):
        pltpu.async_copy(rows, agg_sh.at[id_], semsc, add=True)

    def wait_scatter(rows, id_, semsc):
        pltpu.make_async_copy(rows, agg_sh.at[id_], semsc).wait()

    load_idx(base, idxs0, idxd0)
    start_fetch(base, idxs0, rows0, epbv0, semg0, seme0)

    def pair(p, carry):
        g1 = base + (2 * p + 1) * CHUNK
        g2 = base + (2 * p + 2) * CHUNK

        @pl.when(p > 0)
        def _():
            wait_scatter(rows1, idxd1, semsc1)

        load_idx(g1, idxs1, idxd1)
        start_fetch(g1, idxs1, rows1, epbv1, semg1, seme1)

        wait_fetch(idxs0, rows0, epbv0, semg0, seme0)
        compute(rows0, epbv0)
        start_scatter(rows0, idxd0, semsc0)

        wait_fetch(idxs1, rows1, epbv1, semg1, seme1)
        compute(rows1, epbv1)
        start_scatter(rows1, idxd1, semsc1)

        wait_scatter(rows0, idxd0, semsc0)

        @pl.when(p < cpt // 2 - 1)
        def _():
            load_idx(g2, idxs0, idxd0)
            start_fetch(g2, idxs0, rows0, epbv0, semg0, seme0)
        return carry

    lax.fori_loop(0, cpt // 2, pair, 0)
    wait_scatter(rows1, idxd1, semsc1)

    plsc.subcore_barrier()
    pltpu.sync_copy(agg_sh.at[pl.ds(s * ROWS_PER_TILE, ROWS_PER_TILE)],
                    out_hbm.at[c, pl.ds(s * ROWS_PER_TILE, ROWS_PER_TILE)])


@functools.cache
def _make_sc_agg():
    return functools.partial(
        pl.kernel,
        out_type=jax.ShapeDtypeStruct((NCORES, NPAD, H), jnp.float32),
        mesh=plsc.VectorSubcoreMesh(core_axis_name="c", subcore_axis_name="s"),
        scratch_types=[
            pltpu.VMEM((32, H), jnp.float32),
            pltpu.VMEM((CHUNK,), jnp.int32),
            pltpu.VMEM((CHUNK,), jnp.int32),
            pltpu.VMEM((CHUNK,), jnp.int32),
            pltpu.VMEM((CHUNK,), jnp.int32),
            pltpu.VMEM((CHUNK, H), jnp.float32),
            pltpu.VMEM((CHUNK, H), jnp.float32),
            pltpu.VMEM((CHUNK, H), jnp.float32),
            pltpu.VMEM((CHUNK, H), jnp.float32),
            pltpu.VMEM_SHARED((NPAD, H), jnp.float32),
            pltpu.SemaphoreType.DMA,
            pltpu.SemaphoreType.DMA,
            pltpu.SemaphoreType.DMA,
            pltpu.SemaphoreType.DMA,
            pltpu.SemaphoreType.DMA,
            pltpu.SemaphoreType.DMA,
            pltpu.SemaphoreType.DMA,
        ],
    )(_sc_agg_body)


def kernel(x, edge_index, edge_attr, batch, W_enc, b_enc, W_msg, W_edge, b_msg,
           W_upd, W_self, b_upd, W_dec, b_dec):
    f32 = jnp.float32
    pad = EPAD - E
    src_p = jnp.concatenate([edge_index[0], jnp.zeros((pad,), jnp.int32)])
    dst_p = jnp.concatenate([edge_index[1], jnp.full((pad,), N, jnp.int32)])
    ea_p = jnp.concatenate([edge_attr, jnp.zeros((EA_PAD - E, DE), f32)])

    Wmx, Wmh = W_msg[:H], W_msg[H:2 * H] + W_msg[2 * H:]
    Wsx, Wsh = W_self[:H], W_self[H:2 * H] + W_self[2 * H:]
    Wdx, Wdh = W_dec[:H], W_dec[H:]
    be, bm = b_enc.reshape(1, H), b_msg.reshape(1, H)
    bu, bd = b_upd.reshape(1, H), b_dec.reshape(1, D)

    xin, hpx, hp, sxb = _tc_pre(x, W_enc, be, Wmx, Wmh, Wsx, bu)
    epb = _tc_epb(ea_p, W_edge, bm)

    sc_agg = _make_sc_agg()
    h = xin
    for _ in range(T - 1):
        aggp = sc_agg(hp, src_p, dst_p, epb)
        h, hp = _tc_step(aggp, h, hpx, sxb, W_upd, Wsh, Wmh)

    aggp = sc_agg(hp, src_p, dst_p, epb)
    h, out = _tc_last(aggp, h, xin, sxb, W_upd, Wsh, Wdx, Wdh, bd)
    return (out, h)


# speedup vs baseline: 1.4339x; 1.0317x over previous
import functools

import jax
import jax.numpy as jnp
from jax import lax
from jax.experimental import pallas as pl
from jax.experimental.pallas import tpu as pltpu
from jax.experimental.pallas import tpu_sc as plsc

N, E, D, H, DE, T = 10000, 320000, 128, 128, 16, 4

NPAD = 10240
CHUNK = 64
GROUP = 16
NCORES, NSUB = 2, 16
NTILES = NCORES * NSUB
NC0, NC1 = 190, 126
EPAD = NSUB * (NC0 + NC1) * CHUNK
ROWS_PER_TILE = NPAD // NSUB
RB = 1000
EB = 3200
NEPB = 326400

def _dot(a, b):
    return jnp.dot(a, b, preferred_element_type=jnp.float32)


def _tc_pre_body(x_ref, we_ref, be_ref, wmx_ref, wmh_ref, wsx_ref, bu_ref,
                 xin_ref, hpx_ref, hp_ref, sxb_ref):
    xin = jnp.maximum(_dot(x_ref[...], we_ref[...]) + be_ref[...], 0.0)
    xin_ref[...] = xin
    hpx = _dot(xin, wmx_ref[...])
    hpx_ref[...] = hpx
    hp_ref[...] = hpx + _dot(xin, wmh_ref[...])
    sxb_ref[...] = _dot(xin, wsx_ref[...]) + bu_ref[...]


def _tc_pre(x, We, be, Wmx, Wmh, Wsx, bu):
    wspec = pl.BlockSpec((D, H), lambda i: (0, 0))
    bspec = pl.BlockSpec((1, H), lambda i: (0, 0))
    rspec = pl.BlockSpec((RB, D), lambda i: (i, 0))
    ospec = pl.BlockSpec((RB, H), lambda i: (i, 0))
    return pl.pallas_call(
        _tc_pre_body,
        grid=(N // RB,),
        in_specs=[rspec, wspec, bspec, wspec, wspec, wspec, bspec],
        out_specs=[ospec] * 4,
        out_shape=[jax.ShapeDtypeStruct((N, H), jnp.float32)] * 4,
    )(x, We, be, Wmx, Wmh, Wsx, bu)


def _tc_epb_body(ea_ref, we_ref, bm_ref, epb_ref):
    epb_ref[...] = _dot(ea_ref[...], we_ref[...]) + bm_ref[...]


def _tc_epb(ea, W_edge, bm):
    return pl.pallas_call(
        _tc_epb_body,
        grid=(E // EB,),
        in_specs=[pl.BlockSpec((EB, DE), lambda i: (i, 0)),
                  pl.BlockSpec((DE, H), lambda i: (0, 0)),
                  pl.BlockSpec((1, H), lambda i: (0, 0))],
        out_specs=pl.BlockSpec((EB, H), lambda i: (i, 0)),
        out_shape=jax.ShapeDtypeStruct((NEPB, H), jnp.float32),
    )(ea, W_edge, bm)


def _tc_step_body(aggp_ref, h_ref, hpx_ref, sxb_ref, wu_ref, wsh_ref, wmh_ref,
                  h2_ref, hp2_ref):
    agg = aggp_ref[0] + aggp_ref[1]
    h2 = jnp.maximum(
        _dot(agg, wu_ref[...]) + _dot(h_ref[...], wsh_ref[...]) + sxb_ref[...],
        0.0)
    h2_ref[...] = h2
    hp2_ref[...] = hpx_ref[...] + _dot(h2, wmh_ref[...])


def _tc_step(aggp, h, hpx, sxb, W_upd, Wsh, Wmh):
    wspec = pl.BlockSpec((H, H), lambda i: (0, 0))
    rspec = pl.BlockSpec((RB, H), lambda i: (i, 0))
    return pl.pallas_call(
        _tc_step_body,
        grid=(N // RB,),
        in_specs=[pl.BlockSpec((NCORES, RB, H), lambda i: (0, i, 0)),
                  rspec, rspec, rspec, wspec, wspec, wspec],
        out_specs=[rspec, rspec],
        out_shape=[jax.ShapeDtypeStruct((N, H), jnp.float32)] * 2,
    )(aggp, h, hpx, sxb, W_upd, Wsh, Wmh)


def _tc_last_body(aggp_ref, h_ref, xin_ref, sxb_ref, wu_ref, wsh_ref,
                  wdx_ref, wdh_ref, bd_ref, h2_ref, out_ref):
    agg = aggp_ref[0] + aggp_ref[1]
    h2 = jnp.maximum(
        _dot(agg, wu_ref[...]) + _dot(h_ref[...], wsh_ref[...]) + sxb_ref[...],
        0.0)
    h2_ref[...] = h2
    out_ref[...] = (_dot(xin_ref[...], wdx_ref[...]) +
                    _dot(h2, wdh_ref[...]) + bd_ref[...])


def _tc_last(aggp, h, xin, sxb, W_upd, Wsh, Wdx, Wdh, bd):
    wspec = pl.BlockSpec((H, H), lambda i: (0, 0))
    rspec = pl.BlockSpec((RB, H), lambda i: (i, 0))
    return pl.pallas_call(
        _tc_last_body,
        grid=(N // RB,),
        in_specs=[pl.BlockSpec((NCORES, RB, H), lambda i: (0, i, 0)),
                  rspec, rspec, rspec, wspec, wspec, wspec, wspec,
                  pl.BlockSpec((1, D), lambda i: (0, 0))],
        out_specs=[rspec, pl.BlockSpec((RB, D), lambda i: (i, 0))],
        out_shape=[jax.ShapeDtypeStruct((N, H), jnp.float32),
                   jax.ShapeDtypeStruct((N, D), jnp.float32)],
    )(aggp, h, xin, sxb, W_upd, Wsh, Wdx, Wdh, bd)


def _sc_agg_body(hp_hbm, src_hbm, dst_hbm, epb_hbm, out_hbm,
                 zbuf, idxs0, idxd0, idxs1, idxd1,
                 rows0, epbv0, rows1, epbv1, agg_sh,
                 semg0, seme0, semsc0, semg1, seme1, semsc1, semz):
    c = lax.axis_index("c")
    s = lax.axis_index("s")

    def _z(j, carry):
        for l in range(H // 16):
            zbuf[j, pl.ds(l * 16, 16)] = jnp.zeros((16,), jnp.float32)
        return carry
    lax.fori_loop(0, 32, _z, 0)

    def _zs(k, carry):
        pltpu.async_copy(zbuf, agg_sh.at[pl.ds(s * ROWS_PER_TILE + k * 32, 32)],
                         semz)
        return carry
    lax.fori_loop(0, ROWS_PER_TILE // 32, _zs, 0)

    def _zw(k, carry):
        pltpu.make_async_copy(zbuf, agg_sh.at[pl.ds(s * ROWS_PER_TILE, 32)],
                              semz).wait()
        return carry
    lax.fori_loop(0, ROWS_PER_TILE // 32, _zw, 0)
    plsc.subcore_barrier()

    cpt = jnp.where(c == 0, NC0, NC1)
    base = c * (NSUB * NC0 * CHUNK) + s * cpt * CHUNK

    def load_idx(eb, is_, id_):
        pltpu.sync_copy(src_hbm.at[pl.ds(eb, CHUNK)], is_)
        pltpu.sync_copy(dst_hbm.at[pl.ds(eb, CHUNK)], id_)

    def start_fetch(eb, is_, rows, epbv, semg, seme):
        pltpu.async_copy(hp_hbm.at[is_], rows, semg)
        pltpu.async_copy(epb_hbm.at[pl.ds(eb, CHUNK)], epbv, seme)

    def wait_fetch(is_, rows, epbv, semg, seme):
        pltpu.make_async_copy(hp_hbm.at[is_], rows, semg).wait()
        pltpu.make_async_copy(epb_hbm.at[pl.ds(0, CHUNK)], epbv, seme).wait()

    def compute(r, e):
        @plsc.parallel_loop(0, CHUNK)
        def _(i):
            for l in range(H // 16):
                sl = pl.ds(l * 16, 16)
                r[i, sl] = jnp.maximum(r[i, sl] + e[i, sl], 0.0)

    def start_scatter(rows, id_, semsc):
        pltpu.async_copy(rows, agg_sh.at[id_], semsc, add=True)

    def wait_scatter(rows, id_, semsc):
        pltpu.make_async_copy(rows, agg_sh.at[id_], semsc).wait()

    load_idx(base, idxs0, idxd0)
    start_fetch(base, idxs0, rows0, epbv0, semg0, seme0)

    def pair(p, carry):
        g1 = base + (2 * p + 1) * CHUNK
        g2 = base + (2 * p + 2) * CHUNK

        @pl.when(p > 0)
        def _():
            wait_scatter(rows1, idxd1, semsc1)

        load_idx(g1, idxs1, idxd1)
        start_fetch(g1, idxs1, rows1, epbv1, semg1, seme1)

        wait_fetch(idxs0, rows0, epbv0, semg0, seme0)
        compute(rows0, epbv0)
        start_scatter(rows0, idxd0, semsc0)

        wait_fetch(idxs1, rows1, epbv1, semg1, seme1)
        compute(rows1, epbv1)
        start_scatter(rows1, idxd1, semsc1)

        wait_scatter(rows0, idxd0, semsc0)

        @pl.when(p < cpt // 2 - 1)
        def _():
            load_idx(g2, idxs0, idxd0)
            start_fetch(g2, idxs0, rows0, epbv0, semg0, seme0)
        return carry

    lax.fori_loop(0, cpt // 2, pair, 0)
    wait_scatter(rows1, idxd1, semsc1)

    plsc.subcore_barrier()
    pltpu.sync_copy(agg_sh.at[pl.ds(s * ROWS_PER_TILE, ROWS_PER_TILE)],
                    out_hbm.at[c, pl.ds(s * ROWS_PER_TILE, ROWS_PER_TILE)])


@functools.cache
def _make_sc_agg():
    return functools.partial(
        pl.kernel,
        out_type=jax.ShapeDtypeStruct((NCORES, NPAD, H), jnp.float32),
        mesh=plsc.VectorSubcoreMesh(core_axis_name="c", subcore_axis_name="s"),
        scratch_types=[
            pltpu.VMEM((32, H), jnp.float32),
            pltpu.VMEM((CHUNK,), jnp.int32),
            pltpu.VMEM((CHUNK,), jnp.int32),
            pltpu.VMEM((CHUNK,), jnp.int32),
            pltpu.VMEM((CHUNK,), jnp.int32),
            pltpu.VMEM((CHUNK, H), jnp.float32),
            pltpu.VMEM((CHUNK, H), jnp.float32),
            pltpu.VMEM((CHUNK, H), jnp.float32),
            pltpu.VMEM((CHUNK, H), jnp.float32),
            pltpu.VMEM_SHARED((NPAD, H), jnp.float32),
            pltpu.SemaphoreType.DMA,
            pltpu.SemaphoreType.DMA,
            pltpu.SemaphoreType.DMA,
            pltpu.SemaphoreType.DMA,
            pltpu.SemaphoreType.DMA,
            pltpu.SemaphoreType.DMA,
            pltpu.SemaphoreType.DMA,
        ],
    )(_sc_agg_body)


def kernel(x, edge_index, edge_attr, batch, W_enc, b_enc, W_msg, W_edge, b_msg,
           W_upd, W_self, b_upd, W_dec, b_dec):
    f32 = jnp.float32
    pad = EPAD - E
    src_p = jnp.concatenate([edge_index[0], jnp.zeros((pad,), jnp.int32)])
    dst_p = jnp.concatenate([edge_index[1], jnp.full((pad,), N, jnp.int32)])

    Wmx, Wmh = W_msg[:H], W_msg[H:2 * H] + W_msg[2 * H:]
    Wsx, Wsh = W_self[:H], W_self[H:2 * H] + W_self[2 * H:]
    Wdx, Wdh = W_dec[:H], W_dec[H:]
    be, bm = b_enc.reshape(1, H), b_msg.reshape(1, H)
    bu, bd = b_upd.reshape(1, H), b_dec.reshape(1, D)

    xin, hpx, hp, sxb = _tc_pre(x, W_enc, be, Wmx, Wmh, Wsx, bu)
    epb = _tc_epb(edge_attr, W_edge, bm)

    sc_agg = _make_sc_agg()
    h = xin
    for _ in range(T - 1):
        aggp = sc_agg(hp, src_p, dst_p, epb)
        h, hp = _tc_step(aggp, h, hpx, sxb, W_upd, Wsh, Wmh)

    aggp = sc_agg(hp, src_p, dst_p, epb)
    h, out = _tc_last(aggp, h, xin, sxb, W_upd, Wsh, Wdx, Wdh, bd)
    return (out, h)


# speedup vs baseline: 1.6495x; 1.1503x over previous
import functools

import jax
import jax.numpy as jnp
from jax import lax
from jax.experimental import pallas as pl
from jax.experimental.pallas import tpu as pltpu
from jax.experimental.pallas import tpu_sc as plsc

N, E, D, H, DE, T = 10000, 320000, 128, 128, 16, 4

NPAD = 10240
CHUNK = 64
GROUP = 16
NCORES, NSUB = 2, 16
NTILES = NCORES * NSUB
NC0, NC1 = 190, 126
EPAD = NSUB * (NC0 + NC1) * CHUNK
ROWS_PER_TILE = NPAD // NSUB
RB = 1000
EB = 3200
NEPB = 326400

def _dot(a, b):
    return jnp.dot(a, b, preferred_element_type=jnp.float32)


def _tc_pre_body(x_ref, we_ref, be_ref, wmx_ref, wmh_ref, wsx_ref, bu_ref,
                 xin_ref, hpx_ref, hp_ref, sxb_ref):
    xin = jnp.maximum(_dot(x_ref[...], we_ref[...]) + be_ref[...], 0.0)
    xin_ref[...] = xin
    hpx = _dot(xin, wmx_ref[...])
    hpx_ref[...] = hpx
    hp_ref[...] = hpx + _dot(xin, wmh_ref[...])
    sxb_ref[...] = _dot(xin, wsx_ref[...]) + bu_ref[...]


def _tc_pre(x, We, be, Wmx, Wmh, Wsx, bu):
    wspec = pl.BlockSpec((D, H), lambda i: (0, 0))
    bspec = pl.BlockSpec((1, H), lambda i: (0, 0))
    rspec = pl.BlockSpec((RB, D), lambda i: (i, 0))
    ospec = pl.BlockSpec((RB, H), lambda i: (i, 0))
    return pl.pallas_call(
        _tc_pre_body,
        grid=(N // RB,),
        in_specs=[rspec, wspec, bspec, wspec, wspec, wspec, bspec],
        out_specs=[ospec] * 4,
        out_shape=[jax.ShapeDtypeStruct((N, H), jnp.float32)] * 4,
    )(x, We, be, Wmx, Wmh, Wsx, bu)


def _tc_epb_body(ea_ref, we_ref, bm_ref, epb_ref):
    epb_ref[...] = _dot(ea_ref[...], we_ref[...]) + bm_ref[...]


def _tc_epb(ea, W_edge, bm):
    return pl.pallas_call(
        _tc_epb_body,
        grid=(E // EB,),
        in_specs=[pl.BlockSpec((EB, DE), lambda i: (i, 0)),
                  pl.BlockSpec((DE, H), lambda i: (0, 0)),
                  pl.BlockSpec((1, H), lambda i: (0, 0))],
        out_specs=pl.BlockSpec((EB, H), lambda i: (i, 0)),
        out_shape=jax.ShapeDtypeStruct((NEPB, H), jnp.float32),
    )(ea, W_edge, bm)


def _tc_step_body(aggp_ref, h_ref, hpx_ref, sxb_ref, wu_ref, wsh_ref, wmh_ref,
                  h2_ref, hp2_ref):
    agg = aggp_ref[0] + aggp_ref[1]
    h2 = jnp.maximum(
        _dot(agg, wu_ref[...]) + _dot(h_ref[...], wsh_ref[...]) + sxb_ref[...],
        0.0)
    h2_ref[...] = h2
    hp2_ref[...] = hpx_ref[...] + _dot(h2, wmh_ref[...])


def _tc_step(aggp, h, hpx, sxb, W_upd, Wsh, Wmh):
    wspec = pl.BlockSpec((H, H), lambda i: (0, 0))
    rspec = pl.BlockSpec((RB, H), lambda i: (i, 0))
    return pl.pallas_call(
        _tc_step_body,
        grid=(N // RB,),
        in_specs=[pl.BlockSpec((NCORES, RB, H), lambda i: (0, i, 0)),
                  rspec, rspec, rspec, wspec, wspec, wspec],
        out_specs=[rspec, rspec],
        out_shape=[jax.ShapeDtypeStruct((N, H), jnp.float32)] * 2,
    )(aggp, h, hpx, sxb, W_upd, Wsh, Wmh)


def _tc_last_body(aggp_ref, h_ref, xin_ref, sxb_ref, wu_ref, wsh_ref,
                  wdx_ref, wdh_ref, bd_ref, h2_ref, out_ref):
    agg = aggp_ref[0] + aggp_ref[1]
    h2 = jnp.maximum(
        _dot(agg, wu_ref[...]) + _dot(h_ref[...], wsh_ref[...]) + sxb_ref[...],
        0.0)
    h2_ref[...] = h2
    out_ref[...] = (_dot(xin_ref[...], wdx_ref[...]) +
                    _dot(h2, wdh_ref[...]) + bd_ref[...])


def _tc_last(aggp, h, xin, sxb, W_upd, Wsh, Wdx, Wdh, bd):
    wspec = pl.BlockSpec((H, H), lambda i: (0, 0))
    rspec = pl.BlockSpec((RB, H), lambda i: (i, 0))
    return pl.pallas_call(
        _tc_last_body,
        grid=(N // RB,),
        in_specs=[pl.BlockSpec((NCORES, RB, H), lambda i: (0, i, 0)),
                  rspec, rspec, rspec, wspec, wspec, wspec, wspec,
                  pl.BlockSpec((1, D), lambda i: (0, 0))],
        out_specs=[rspec, pl.BlockSpec((RB, D), lambda i: (i, 0))],
        out_shape=[jax.ShapeDtypeStruct((N, H), jnp.float32),
                   jax.ShapeDtypeStruct((N, D), jnp.float32)],
    )(aggp, h, xin, sxb, W_upd, Wsh, Wdx, Wdh, bd)


def _sc_agg_body(hp_hbm, src_hbm, dst_hbm, epb_hbm, out_hbm,
                 zbuf, idxs0, idxd0, idxs1, idxd1, sidx0, sidx1,
                 rows0, epbv0, rows1, epbv1, agg_sh,
                 semg0, seme0, semsc0, semg1, seme1, semsc1, semz):
    c = lax.axis_index("c")
    s = lax.axis_index("s")

    def _z(j, carry):
        for l in range(H // 16):
            zbuf[j, pl.ds(l * 16, 16)] = jnp.zeros((16,), jnp.float32)
        return carry
    lax.fori_loop(0, 32, _z, 0)

    def _zs(k, carry):
        pltpu.async_copy(zbuf, agg_sh.at[pl.ds(s * ROWS_PER_TILE + k * 32, 32)],
                         semz)
        return carry
    lax.fori_loop(0, ROWS_PER_TILE // 32, _zs, 0)

    def _zw(k, carry):
        pltpu.make_async_copy(zbuf, agg_sh.at[pl.ds(s * ROWS_PER_TILE, 32)],
                              semz).wait()
        return carry
    lax.fori_loop(0, ROWS_PER_TILE // 32, _zw, 0)
    plsc.subcore_barrier()

    cpt = jnp.where(c == 0, NC0, NC1)
    base = c * (NSUB * NC0 * CHUNK) + s * cpt * CHUNK

    def load_idx(eb, is_, id_):
        pltpu.sync_copy(src_hbm.at[pl.ds(eb, CHUNK)], is_)
        pltpu.sync_copy(dst_hbm.at[pl.ds(eb, CHUNK)], id_)

    def start_gather(is_, rows, semg):
        pltpu.async_copy(hp_hbm.at[is_], rows, semg)

    def start_epb(eb, epbv, seme):
        pltpu.async_copy(epb_hbm.at[pl.ds(eb, CHUNK)], epbv, seme)

    def wait_gather(is_, rows, semg):
        pltpu.make_async_copy(hp_hbm.at[is_], rows, semg).wait()

    def wait_epb(epbv, seme):
        pltpu.make_async_copy(epb_hbm.at[pl.ds(0, CHUNK)], epbv, seme).wait()

    def compute(r, e):
        @plsc.parallel_loop(0, CHUNK)
        def _(i):
            for l in range(H // 16):
                sl = pl.ds(l * 16, 16)
                e[i, sl] = jnp.maximum(r[i, sl] + e[i, sl], 0.0)

    def start_scatter(id_, sidx, m, semsc):
        for l in range(CHUNK // 16):
            sl = pl.ds(l * 16, 16)
            sidx[sl] = id_[sl]
        pltpu.async_copy(m, agg_sh.at[sidx], semsc, add=True)

    def wait_scatter(sidx, m, semsc):
        pltpu.make_async_copy(m, agg_sh.at[sidx], semsc).wait()

    load_idx(base, idxs0, idxd0)
    start_gather(idxs0, rows0, semg0)
    start_epb(base, epbv0, seme0)

    def pair(p, carry):
        g1 = base + (2 * p + 1) * CHUNK
        g2 = base + (2 * p + 2) * CHUNK

        load_idx(g1, idxs1, idxd1)
        start_gather(idxs1, rows1, semg1)

        @pl.when(p > 0)
        def _():
            wait_scatter(sidx1, epbv1, semsc1)
        start_epb(g1, epbv1, seme1)

        wait_gather(idxs0, rows0, semg0)
        wait_epb(epbv0, seme0)
        compute(rows0, epbv0)
        start_scatter(idxd0, sidx0, epbv0, semsc0)

        @pl.when(p < cpt // 2 - 1)
        def _():
            load_idx(g2, idxs0, idxd0)
            start_gather(idxs0, rows0, semg0)

        wait_gather(idxs1, rows1, semg1)
        wait_epb(epbv1, seme1)
        compute(rows1, epbv1)
        start_scatter(idxd1, sidx1, epbv1, semsc1)

        @pl.when(p < cpt // 2 - 1)
        def _():
            wait_scatter(sidx0, epbv0, semsc0)
            start_epb(g2, epbv0, seme0)
        return carry

    lax.fori_loop(0, cpt // 2, pair, 0)
    wait_scatter(sidx0, epbv0, semsc0)
    wait_scatter(sidx1, epbv1, semsc1)

    plsc.subcore_barrier()
    pltpu.sync_copy(agg_sh.at[pl.ds(s * ROWS_PER_TILE, ROWS_PER_TILE)],
                    out_hbm.at[c, pl.ds(s * ROWS_PER_TILE, ROWS_PER_TILE)])


@functools.cache
def _make_sc_agg():
    return functools.partial(
        pl.kernel,
        out_type=jax.ShapeDtypeStruct((NCORES, NPAD, H), jnp.float32),
        mesh=plsc.VectorSubcoreMesh(core_axis_name="c", subcore_axis_name="s"),
        scratch_types=[
            pltpu.VMEM((32, H), jnp.float32),
            pltpu.VMEM((CHUNK,), jnp.int32),
            pltpu.VMEM((CHUNK,), jnp.int32),
            pltpu.VMEM((CHUNK,), jnp.int32),
            pltpu.VMEM((CHUNK,), jnp.int32),
            pltpu.VMEM((CHUNK,), jnp.int32),
            pltpu.VMEM((CHUNK,), jnp.int32),
            pltpu.VMEM((CHUNK, H), jnp.float32),
            pltpu.VMEM((CHUNK, H), jnp.float32),
            pltpu.VMEM((CHUNK, H), jnp.float32),
            pltpu.VMEM((CHUNK, H), jnp.float32),
            pltpu.VMEM_SHARED((NPAD, H), jnp.float32),
            pltpu.SemaphoreType.DMA,
            pltpu.SemaphoreType.DMA,
            pltpu.SemaphoreType.DMA,
            pltpu.SemaphoreType.DMA,
            pltpu.SemaphoreType.DMA,
            pltpu.SemaphoreType.DMA,
            pltpu.SemaphoreType.DMA,
        ],
    )(_sc_agg_body)


def kernel(x, edge_index, edge_attr, batch, W_enc, b_enc, W_msg, W_edge, b_msg,
           W_upd, W_self, b_upd, W_dec, b_dec):
    f32 = jnp.float32
    pad = EPAD - E
    src_p = jnp.concatenate([edge_index[0], jnp.zeros((pad,), jnp.int32)])
    dst_p = jnp.concatenate([edge_index[1], jnp.full((pad,), N, jnp.int32)])

    Wmx, Wmh = W_msg[:H], W_msg[H:2 * H] + W_msg[2 * H:]
    Wsx, Wsh = W_self[:H], W_self[H:2 * H] + W_self[2 * H:]
    Wdx, Wdh = W_dec[:H], W_dec[H:]
    be, bm = b_enc.reshape(1, H), b_msg.reshape(1, H)
    bu, bd = b_upd.reshape(1, H), b_dec.reshape(1, D)

    xin, hpx, hp, sxb = _tc_pre(x, W_enc, be, Wmx, Wmh, Wsx, bu)
    epb = _tc_epb(edge_attr, W_edge, bm)

    sc_agg = _make_sc_agg()
    h = xin
    for _ in range(T - 1):
        aggp = sc_agg(hp, src_p, dst_p, epb)
        h, hp = _tc_step(aggp, h, hpx, sxb, W_upd, Wsh, Wmh)

    aggp = sc_agg(hp, src_p, dst_p, epb)
    h, out = _tc_last(aggp, h, xin, sxb, W_upd, Wsh, Wdx, Wdh, bd)
    return (out, h)


# speedup vs baseline: 1.6936x; 1.0267x over previous
import functools

import jax
import jax.numpy as jnp
from jax import lax
from jax.experimental import pallas as pl
from jax.experimental.pallas import tpu as pltpu
from jax.experimental.pallas import tpu_sc as plsc

N, E, D, H, DE, T = 10000, 320000, 128, 128, 16, 4

NPAD = 10240
CHUNK = 64
GROUP = 16
NCORES, NSUB = 2, 16
NTILES = NCORES * NSUB
NC0, NC1 = 190, 126
EPAD = NSUB * (NC0 + NC1) * CHUNK
ROWS_PER_TILE = NPAD // NSUB
RB = 1000
EB = 3200
NEPB = 326400

def _dot(a, b):
    return jnp.dot(a, b, preferred_element_type=jnp.float32)


def _tc_pre_body(x_ref, we_ref, be_ref, wmx_ref, wmh_ref, wsx_ref, bu_ref,
                 xin_ref, hpx_ref, hp_ref, sxb_ref):
    xin = jnp.maximum(_dot(x_ref[...], we_ref[...]) + be_ref[...], 0.0)
    xin_ref[...] = xin
    hpx = _dot(xin, wmx_ref[...])
    hpx_ref[...] = hpx
    hp_ref[...] = hpx + _dot(xin, wmh_ref[...])
    sxb_ref[...] = _dot(xin, wsx_ref[...]) + bu_ref[...]


def _tc_pre(x, We, be, Wmx, Wmh, Wsx, bu):
    wspec = pl.BlockSpec((D, H), lambda i: (0, 0))
    bspec = pl.BlockSpec((1, H), lambda i: (0, 0))
    rspec = pl.BlockSpec((RB, D), lambda i: (i, 0))
    ospec = pl.BlockSpec((RB, H), lambda i: (i, 0))
    return pl.pallas_call(
        _tc_pre_body,
        grid=(N // RB,),
        in_specs=[rspec, wspec, bspec, wspec, wspec, wspec, bspec],
        out_specs=[ospec] * 4,
        out_shape=[jax.ShapeDtypeStruct((N, H), jnp.float32)] * 4,
    )(x, We, be, Wmx, Wmh, Wsx, bu)


def _tc_epb_body(ea_ref, we_ref, bm_ref, epb_ref):
    epb_ref[...] = _dot(ea_ref[...], we_ref[...]) + bm_ref[...]


def _tc_epb(ea, W_edge, bm):
    return pl.pallas_call(
        _tc_epb_body,
        grid=(E // EB,),
        in_specs=[pl.BlockSpec((EB, DE), lambda i: (i, 0)),
                  pl.BlockSpec((DE, H), lambda i: (0, 0)),
                  pl.BlockSpec((1, H), lambda i: (0, 0))],
        out_specs=pl.BlockSpec((EB, H), lambda i: (i, 0)),
        out_shape=jax.ShapeDtypeStruct((NEPB, H), jnp.float32),
    )(ea, W_edge, bm)


def _tc_step_body(aggp_ref, h_ref, hpx_ref, sxb_ref, wu_ref, wsh_ref, wmh_ref,
                  h2_ref, hp2_ref):
    agg = aggp_ref[0] + aggp_ref[1]
    h2 = jnp.maximum(
        _dot(agg, wu_ref[...]) + _dot(h_ref[...], wsh_ref[...]) + sxb_ref[...],
        0.0)
    h2_ref[...] = h2
    hp2_ref[...] = hpx_ref[...] + _dot(h2, wmh_ref[...])


def _tc_step(aggp, h, hpx, sxb, W_upd, Wsh, Wmh):
    wspec = pl.BlockSpec((H, H), lambda i: (0, 0))
    rspec = pl.BlockSpec((RB, H), lambda i: (i, 0))
    return pl.pallas_call(
        _tc_step_body,
        grid=(N // RB,),
        in_specs=[pl.BlockSpec((NCORES, RB, H), lambda i: (0, i, 0)),
                  rspec, rspec, rspec, wspec, wspec, wspec],
        out_specs=[rspec, rspec],
        out_shape=[jax.ShapeDtypeStruct((N, H), jnp.float32)] * 2,
    )(aggp, h, hpx, sxb, W_upd, Wsh, Wmh)


def _tc_last_body(aggp_ref, h_ref, xin_ref, sxb_ref, wu_ref, wsh_ref,
                  wdx_ref, wdh_ref, bd_ref, h2_ref, out_ref):
    agg = aggp_ref[0] + aggp_ref[1]
    h2 = jnp.maximum(
        _dot(agg, wu_ref[...]) + _dot(h_ref[...], wsh_ref[...]) + sxb_ref[...],
        0.0)
    h2_ref[...] = h2
    out_ref[...] = (_dot(xin_ref[...], wdx_ref[...]) +
                    _dot(h2, wdh_ref[...]) + bd_ref[...])


def _tc_last(aggp, h, xin, sxb, W_upd, Wsh, Wdx, Wdh, bd):
    wspec = pl.BlockSpec((H, H), lambda i: (0, 0))
    rspec = pl.BlockSpec((RB, H), lambda i: (i, 0))
    return pl.pallas_call(
        _tc_last_body,
        grid=(N // RB,),
        in_specs=[pl.BlockSpec((NCORES, RB, H), lambda i: (0, i, 0)),
                  rspec, rspec, rspec, wspec, wspec, wspec, wspec,
                  pl.BlockSpec((1, D), lambda i: (0, 0))],
        out_specs=[rspec, pl.BlockSpec((RB, D), lambda i: (i, 0))],
        out_shape=[jax.ShapeDtypeStruct((N, H), jnp.float32),
                   jax.ShapeDtypeStruct((N, D), jnp.float32)],
    )(aggp, h, xin, sxb, W_upd, Wsh, Wdx, Wdh, bd)


def _sc_agg_body(hp_hbm, sd_hbm, epb_hbm, out_hbm,
                 zbuf, sdv0, sdv1, gidx0, gidx1, sidx0, sidx1,
                 rows0, epbv0, rows1, epbv1, agg_sh,
                 semg0, seme0, semsc0, semg1, seme1, semsc1, semz):
    c = lax.axis_index("c")
    s = lax.axis_index("s")

    def _z(j, carry):
        for l in range(H // 16):
            zbuf[j, pl.ds(l * 16, 16)] = jnp.zeros((16,), jnp.float32)
        return carry
    lax.fori_loop(0, 32, _z, 0)

    def _zs(k, carry):
        pltpu.async_copy(zbuf, agg_sh.at[pl.ds(s * ROWS_PER_TILE + k * 32, 32)],
                         semz)
        return carry
    lax.fori_loop(0, ROWS_PER_TILE // 32, _zs, 0)

    def _zw(k, carry):
        pltpu.make_async_copy(zbuf, agg_sh.at[pl.ds(s * ROWS_PER_TILE, 32)],
                              semz).wait()
        return carry
    lax.fori_loop(0, ROWS_PER_TILE // 32, _zw, 0)
    plsc.subcore_barrier()

    cpt = jnp.where(c == 0, NC0, NC1)
    cb = c * (NSUB * NC0) + s * cpt
    base = cb * CHUNK

    def load_idx(g, sdv):
        pltpu.sync_copy(sd_hbm.at[cb + g], sdv)

    def start_gather(sdv, gidx, rows, semg):
        for l in range(CHUNK // 16):
            sl = pl.ds(l * 16, 16)
            gidx[sl] = sdv[sl]
        pltpu.async_copy(hp_hbm.at[gidx], rows, semg)

    def start_epb(eb, epbv, seme):
        pltpu.async_copy(epb_hbm.at[pl.ds(eb, CHUNK)], epbv, seme)

    def wait_gather(gidx, rows, semg):
        pltpu.make_async_copy(hp_hbm.at[gidx], rows, semg).wait()

    def wait_epb(epbv, seme):
        pltpu.make_async_copy(epb_hbm.at[pl.ds(0, CHUNK)], epbv, seme).wait()

    def compute(r, e):
        @plsc.parallel_loop(0, CHUNK)
        def _(i):
            for l in range(H // 16):
                sl = pl.ds(l * 16, 16)
                e[i, sl] = jnp.maximum(r[i, sl] + e[i, sl], 0.0)

    def start_scatter(sdv, sidx, m, semsc):
        for l in range(CHUNK // 16):
            sl = pl.ds(l * 16, 16)
            sidx[sl] = sdv[pl.ds(CHUNK + l * 16, 16)]
        pltpu.async_copy(m, agg_sh.at[sidx], semsc, add=True)

    def wait_scatter(sidx, m, semsc):
        pltpu.make_async_copy(m, agg_sh.at[sidx], semsc).wait()

    load_idx(0, sdv0)
    start_gather(sdv0, gidx0, rows0, semg0)
    start_epb(base, epbv0, seme0)

    def pair(p, carry):
        g1e = base + (2 * p + 1) * CHUNK
        g2e = base + (2 * p + 2) * CHUNK

        load_idx(2 * p + 1, sdv1)
        start_gather(sdv1, gidx1, rows1, semg1)

        @pl.when(p > 0)
        def _():
            wait_scatter(sidx1, epbv1, semsc1)
        start_epb(g1e, epbv1, seme1)

        wait_gather(gidx0, rows0, semg0)
        wait_epb(epbv0, seme0)
        compute(rows0, epbv0)
        start_scatter(sdv0, sidx0, epbv0, semsc0)

        @pl.when(p < cpt // 2 - 1)
        def _():
            load_idx(2 * p + 2, sdv0)
            start_gather(sdv0, gidx0, rows0, semg0)

        wait_gather(gidx1, rows1, semg1)
        wait_epb(epbv1, seme1)
        compute(rows1, epbv1)
        start_scatter(sdv1, sidx1, epbv1, semsc1)

        @pl.when(p < cpt // 2 - 1)
        def _():
            wait_scatter(sidx0, epbv0, semsc0)
            start_epb(g2e, epbv0, seme0)
        return carry

    lax.fori_loop(0, cpt // 2, pair, 0)
    wait_scatter(sidx0, epbv0, semsc0)
    wait_scatter(sidx1, epbv1, semsc1)

    plsc.subcore_barrier()
    pltpu.sync_copy(agg_sh.at[pl.ds(s * ROWS_PER_TILE, ROWS_PER_TILE)],
                    out_hbm.at[c, pl.ds(s * ROWS_PER_TILE, ROWS_PER_TILE)])


@functools.cache
def _make_sc_agg():
    return functools.partial(
        pl.kernel,
        out_type=jax.ShapeDtypeStruct((NCORES, NPAD, H), jnp.float32),
        mesh=plsc.VectorSubcoreMesh(core_axis_name="c", subcore_axis_name="s"),
        scratch_types=[
            pltpu.VMEM((32, H), jnp.float32),
            pltpu.VMEM((2 * CHUNK,), jnp.int32),
            pltpu.VMEM((2 * CHUNK,), jnp.int32),
            pltpu.VMEM((CHUNK,), jnp.int32),
            pltpu.VMEM((CHUNK,), jnp.int32),
            pltpu.VMEM((CHUNK,), jnp.int32),
            pltpu.VMEM((CHUNK,), jnp.int32),
            pltpu.VMEM((CHUNK, H), jnp.float32),
            pltpu.VMEM((CHUNK, H), jnp.float32),
            pltpu.VMEM((CHUNK, H), jnp.float32),
            pltpu.VMEM((CHUNK, H), jnp.float32),
            pltpu.VMEM_SHARED((NPAD, H), jnp.float32),
            pltpu.SemaphoreType.DMA,
            pltpu.SemaphoreType.DMA,
            pltpu.SemaphoreType.DMA,
            pltpu.SemaphoreType.DMA,
            pltpu.SemaphoreType.DMA,
            pltpu.SemaphoreType.DMA,
            pltpu.SemaphoreType.DMA,
        ],
    )(_sc_agg_body)


def kernel(x, edge_index, edge_attr, batch, W_enc, b_enc, W_msg, W_edge, b_msg,
           W_upd, W_self, b_upd, W_dec, b_dec):
    f32 = jnp.float32
    pad = EPAD - E
    src_p = jnp.concatenate([edge_index[0], jnp.zeros((pad,), jnp.int32)])
    dst_p = jnp.concatenate([edge_index[1], jnp.full((pad,), N, jnp.int32)])
    sd = jnp.concatenate([src_p.reshape(-1, CHUNK), dst_p.reshape(-1, CHUNK)],
                         axis=1)

    Wmx, Wmh = W_msg[:H], W_msg[H:2 * H] + W_msg[2 * H:]
    Wsx, Wsh = W_self[:H], W_self[H:2 * H] + W_self[2 * H:]
    Wdx, Wdh = W_dec[:H], W_dec[H:]
    be, bm = b_enc.reshape(1, H), b_msg.reshape(1, H)
    bu, bd = b_upd.reshape(1, H), b_dec.reshape(1, D)

    xin, hpx, hp, sxb = _tc_pre(x, W_enc, be, Wmx, Wmh, Wsx, bu)
    epb = _tc_epb(edge_attr, W_edge, bm)

    sc_agg = _make_sc_agg()
    h = xin
    for _ in range(T - 1):
        aggp = sc_agg(hp, sd, epb)
        h, hp = _tc_step(aggp, h, hpx, sxb, W_upd, Wsh, Wmh)

    aggp = sc_agg(hp, sd, epb)
    h, out = _tc_last(aggp, h, xin, sxb, W_upd, Wsh, Wdx, Wdh, bd)
    return (out, h)


# speedup vs baseline: 1.7443x; 1.0299x over previous
import functools

import jax
import jax.numpy as jnp
from jax import lax
from jax.experimental import pallas as pl
from jax.experimental.pallas import tpu as pltpu
from jax.experimental.pallas import tpu_sc as plsc

N, E, D, H, DE, T = 10000, 320000, 128, 128, 16, 4

NPAD = 10240
CHUNK = 64
GROUP = 16
NCORES, NSUB = 2, 16
NTILES = NCORES * NSUB
NC0, NC1 = 198, 118
EPAD = NSUB * (NC0 + NC1) * CHUNK
ROWS_PER_TILE = NPAD // NSUB
RB = 2000
EB = 3200
NEPB = 326400

def _dot(a, b):
    return jnp.dot(a, b, preferred_element_type=jnp.float32)


def _tc_pre_body(x_ref, we_ref, be_ref, wmx_ref, wmh_ref, wsx_ref, bu_ref,
                 xin_ref, hpx_ref, hp_ref, sxb_ref):
    xin = jnp.maximum(_dot(x_ref[...], we_ref[...]) + be_ref[...], 0.0)
    xin_ref[...] = xin
    hpx = _dot(xin, wmx_ref[...])
    hpx_ref[...] = hpx
    hp_ref[...] = hpx + _dot(xin, wmh_ref[...])
    sxb_ref[...] = _dot(xin, wsx_ref[...]) + bu_ref[...]


def _tc_pre(x, We, be, Wmx, Wmh, Wsx, bu):
    wspec = pl.BlockSpec((D, H), lambda i: (0, 0))
    bspec = pl.BlockSpec((1, H), lambda i: (0, 0))
    rspec = pl.BlockSpec((RB, D), lambda i: (i, 0))
    ospec = pl.BlockSpec((RB, H), lambda i: (i, 0))
    return pl.pallas_call(
        _tc_pre_body,
        grid=(N // RB,),
        in_specs=[rspec, wspec, bspec, wspec, wspec, wspec, bspec],
        out_specs=[ospec] * 4,
        out_shape=[jax.ShapeDtypeStruct((N, H), jnp.float32)] * 4,
    )(x, We, be, Wmx, Wmh, Wsx, bu)


def _tc_epb_body(ea_ref, we_ref, bm_ref, epb_ref):
    epb_ref[...] = _dot(ea_ref[...], we_ref[...]) + bm_ref[...]


def _tc_epb(ea, W_edge, bm):
    return pl.pallas_call(
        _tc_epb_body,
        grid=(E // EB,),
        in_specs=[pl.BlockSpec((EB, DE), lambda i: (i, 0)),
                  pl.BlockSpec((DE, H), lambda i: (0, 0)),
                  pl.BlockSpec((1, H), lambda i: (0, 0))],
        out_specs=pl.BlockSpec((EB, H), lambda i: (i, 0)),
        out_shape=jax.ShapeDtypeStruct((NEPB, H), jnp.float32),
    )(ea, W_edge, bm)


def _tc_step_body(aggp_ref, h_ref, hpx_ref, sxb_ref, wu_ref, wsh_ref, wmh_ref,
                  h2_ref, hp2_ref):
    agg = aggp_ref[0] + aggp_ref[1]
    h2 = jnp.maximum(
        _dot(agg, wu_ref[...]) + _dot(h_ref[...], wsh_ref[...]) + sxb_ref[...],
        0.0)
    h2_ref[...] = h2
    hp2_ref[...] = hpx_ref[...] + _dot(h2, wmh_ref[...])


def _tc_step(aggp, h, hpx, sxb, W_upd, Wsh, Wmh):
    wspec = pl.BlockSpec((H, H), lambda i: (0, 0))
    rspec = pl.BlockSpec((RB, H), lambda i: (i, 0))
    return pl.pallas_call(
        _tc_step_body,
        grid=(N // RB,),
        in_specs=[pl.BlockSpec((NCORES, RB, H), lambda i: (0, i, 0)),
                  rspec, rspec, rspec, wspec, wspec, wspec],
        out_specs=[rspec, rspec],
        out_shape=[jax.ShapeDtypeStruct((N, H), jnp.float32)] * 2,
    )(aggp, h, hpx, sxb, W_upd, Wsh, Wmh)


def _tc_last_body(aggp_ref, h_ref, xin_ref, sxb_ref, wu_ref, wsh_ref,
                  wdx_ref, wdh_ref, bd_ref, h2_ref, out_ref):
    agg = aggp_ref[0] + aggp_ref[1]
    h2 = jnp.maximum(
        _dot(agg, wu_ref[...]) + _dot(h_ref[...], wsh_ref[...]) + sxb_ref[...],
        0.0)
    h2_ref[...] = h2
    out_ref[...] = (_dot(xin_ref[...], wdx_ref[...]) +
                    _dot(h2, wdh_ref[...]) + bd_ref[...])


def _tc_last(aggp, h, xin, sxb, W_upd, Wsh, Wdx, Wdh, bd):
    wspec = pl.BlockSpec((H, H), lambda i: (0, 0))
    rspec = pl.BlockSpec((RB, H), lambda i: (i, 0))
    return pl.pallas_call(
        _tc_last_body,
        grid=(N // RB,),
        in_specs=[pl.BlockSpec((NCORES, RB, H), lambda i: (0, i, 0)),
                  rspec, rspec, rspec, wspec, wspec, wspec, wspec,
                  pl.BlockSpec((1, D), lambda i: (0, 0))],
        out_specs=[rspec, pl.BlockSpec((RB, D), lambda i: (i, 0))],
        out_shape=[jax.ShapeDtypeStruct((N, H), jnp.float32),
                   jax.ShapeDtypeStruct((N, D), jnp.float32)],
    )(aggp, h, xin, sxb, W_upd, Wsh, Wdx, Wdh, bd)


def _sc_agg_body(hp_hbm, sd_hbm, epb_hbm, out_hbm,
                 zbuf, sdv0, sdv1, gidx0, gidx1, sidx0, sidx1,
                 rows0, epbv0, rows1, epbv1, agg_sh,
                 semg0, seme0, semsc0, semg1, seme1, semsc1, semz):
    c = lax.axis_index("c")
    s = lax.axis_index("s")

    def _z(j, carry):
        for l in range(H // 16):
            zbuf[j, pl.ds(l * 16, 16)] = jnp.zeros((16,), jnp.float32)
        return carry
    lax.fori_loop(0, 32, _z, 0)

    def _zs(k, carry):
        pltpu.async_copy(zbuf, agg_sh.at[pl.ds(s * ROWS_PER_TILE + k * 32, 32)],
                         semz)
        return carry
    lax.fori_loop(0, ROWS_PER_TILE // 32, _zs, 0)

    def _zw(k, carry):
        pltpu.make_async_copy(zbuf, agg_sh.at[pl.ds(s * ROWS_PER_TILE, 32)],
                              semz).wait()
        return carry
    lax.fori_loop(0, ROWS_PER_TILE // 32, _zw, 0)
    plsc.subcore_barrier()

    cpt = jnp.where(c == 0, NC0, NC1)
    cb = c * (NSUB * NC0) + s * cpt
    base = cb * CHUNK

    def load_idx(g, sdv):
        pltpu.sync_copy(sd_hbm.at[cb + g], sdv)

    def start_gather(sdv, gidx, rows, semg):
        for l in range(CHUNK // 16):
            sl = pl.ds(l * 16, 16)
            gidx[sl] = sdv[sl]
        pltpu.async_copy(hp_hbm.at[gidx], rows, semg)

    def start_epb(eb, epbv, seme):
        pltpu.async_copy(epb_hbm.at[pl.ds(eb, CHUNK)], epbv, seme)

    def wait_gather(gidx, rows, semg):
        pltpu.make_async_copy(hp_hbm.at[gidx], rows, semg).wait()

    def wait_epb(epbv, seme):
        pltpu.make_async_copy(epb_hbm.at[pl.ds(0, CHUNK)], epbv, seme).wait()

    def compute(r, e):
        @plsc.parallel_loop(0, CHUNK)
        def _(i):
            for l in range(H // 16):
                sl = pl.ds(l * 16, 16)
                e[i, sl] = jnp.maximum(r[i, sl] + e[i, sl], 0.0)

    def start_scatter(sdv, sidx, m, semsc):
        for l in range(CHUNK // 16):
            sl = pl.ds(l * 16, 16)
            sidx[sl] = sdv[pl.ds(CHUNK + l * 16, 16)]
        pltpu.async_copy(m, agg_sh.at[sidx], semsc, add=True)

    def wait_scatter(sidx, m, semsc):
        pltpu.make_async_copy(m, agg_sh.at[sidx], semsc).wait()

    load_idx(0, sdv0)
    start_gather(sdv0, gidx0, rows0, semg0)
    start_epb(base, epbv0, seme0)

    def pair(p, carry):
        g1e = base + (2 * p + 1) * CHUNK
        g2e = base + (2 * p + 2) * CHUNK

        load_idx(2 * p + 1, sdv1)
        start_gather(sdv1, gidx1, rows1, semg1)

        @pl.when(p > 0)
        def _():
            wait_scatter(sidx1, epbv1, semsc1)
        start_epb(g1e, epbv1, seme1)

        wait_gather(gidx0, rows0, semg0)
        wait_epb(epbv0, seme0)
        compute(rows0, epbv0)
        start_scatter(sdv0, sidx0, epbv0, semsc0)

        @pl.when(p < cpt // 2 - 1)
        def _():
            load_idx(2 * p + 2, sdv0)
            start_gather(sdv0, gidx0, rows0, semg0)

        wait_gather(gidx1, rows1, semg1)
        wait_epb(epbv1, seme1)
        compute(rows1, epbv1)
        start_scatter(sdv1, sidx1, epbv1, semsc1)

        @pl.when(p < cpt // 2 - 1)
        def _():
            wait_scatter(sidx0, epbv0, semsc0)
            start_epb(g2e, epbv0, seme0)
        return carry

    lax.fori_loop(0, cpt // 2, pair, 0)
    wait_scatter(sidx0, epbv0, semsc0)
    wait_scatter(sidx1, epbv1, semsc1)

    plsc.subcore_barrier()
    pltpu.sync_copy(agg_sh.at[pl.ds(s * ROWS_PER_TILE, ROWS_PER_TILE)],
                    out_hbm.at[c, pl.ds(s * ROWS_PER_TILE, ROWS_PER_TILE)])


@functools.cache
def _make_sc_agg():
    return functools.partial(
        pl.kernel,
        out_type=jax.ShapeDtypeStruct((NCORES, NPAD, H), jnp.float32),
        mesh=plsc.VectorSubcoreMesh(core_axis_name="c", subcore_axis_name="s"),
        scratch_types=[
            pltpu.VMEM((32, H), jnp.float32),
            pltpu.VMEM((2 * CHUNK,), jnp.int32),
            pltpu.VMEM((2 * CHUNK,), jnp.int32),
            pltpu.VMEM((CHUNK,), jnp.int32),
            pltpu.VMEM((CHUNK,), jnp.int32),
            pltpu.VMEM((CHUNK,), jnp.int32),
            pltpu.VMEM((CHUNK,), jnp.int32),
            pltpu.VMEM((CHUNK, H), jnp.float32),
            pltpu.VMEM((CHUNK, H), jnp.float32),
            pltpu.VMEM((CHUNK, H), jnp.float32),
            pltpu.VMEM((CHUNK, H), jnp.float32),
            pltpu.VMEM_SHARED((NPAD, H), jnp.float32),
            pltpu.SemaphoreType.DMA,
            pltpu.SemaphoreType.DMA,
            pltpu.SemaphoreType.DMA,
            pltpu.SemaphoreType.DMA,
            pltpu.SemaphoreType.DMA,
            pltpu.SemaphoreType.DMA,
            pltpu.SemaphoreType.DMA,
        ],
    )(_sc_agg_body)


def kernel(x, edge_index, edge_attr, batch, W_enc, b_enc, W_msg, W_edge, b_msg,
           W_upd, W_self, b_upd, W_dec, b_dec):
    f32 = jnp.float32
    pad = EPAD - E
    src_p = jnp.concatenate([edge_index[0], jnp.zeros((pad,), jnp.int32)])
    dst_p = jnp.concatenate([edge_index[1], jnp.full((pad,), N, jnp.int32)])
    sd = jnp.concatenate([src_p.reshape(-1, CHUNK), dst_p.reshape(-1, CHUNK)],
                         axis=1)

    Wmx, Wmh = W_msg[:H], W_msg[H:2 * H] + W_msg[2 * H:]
    Wsx, Wsh = W_self[:H], W_self[H:2 * H] + W_self[2 * H:]
    Wdx, Wdh = W_dec[:H], W_dec[H:]
    be, bm = b_enc.reshape(1, H), b_msg.reshape(1, H)
    bu, bd = b_upd.reshape(1, H), b_dec.reshape(1, D)

    xin, hpx, hp, sxb = _tc_pre(x, W_enc, be, Wmx, Wmh, Wsx, bu)
    epb = _tc_epb(edge_attr, W_edge, bm)

    sc_agg = _make_sc_agg()
    h = xin
    for _ in range(T - 1):
        aggp = sc_agg(hp, sd, epb)
        h, hp = _tc_step(aggp, h, hpx, sxb, W_upd, Wsh, Wmh)

    aggp = sc_agg(hp, sd, epb)
    h, out = _tc_last(aggp, h, xin, sxb, W_upd, Wsh, Wdx, Wdh, bd)
    return (out, h)
